# Initial kernel scaffold; baseline (speedup 1.0000x reference)
import functools

import jax
import jax.numpy as jnp
from jax import lax
from jax.experimental import pallas as pl
from jax.experimental.pallas import tpu as pltpu
from jax.experimental.pallas import tpu_sc as plsc

N = 10000
E = 320000
D = 128
NPAD = 10016
NW = 32
NS = 16
K = 128
CH = 79
EPW = CH * K
EPAD = NW * EPW

_MESH = dict(core_axis_name="c", subcore_axis_name="s")


@functools.partial(
    pl.kernel,
    out_type=jax.ShapeDtypeStruct((2, NPAD, 8), jnp.float32),
    mesh=plsc.VectorSubcoreMesh(**_MESH),
    scratch_types=[
        pltpu.VMEM((CH, K), jnp.int32),
        pltpu.VMEM((K, 8), jnp.float32),
        pltpu.VMEM_SHARED((NPAD, 8), jnp.float32),
    ],
)
def _sc_degree(dstw_hbm, ones_hbm, zeros_hbm, out_hbm, idx_v, ones_v, deg_sh):
    cid = lax.axis_index("c")
    sid = lax.axis_index("s")
    wid = cid * NS + sid
    pltpu.sync_copy(dstw_hbm.at[wid], idx_v)
    pltpu.sync_copy(ones_hbm, ones_v)

    @pl.when(sid == 0)
    def _():
        pltpu.sync_copy(zeros_hbm, deg_sh)

    plsc.subcore_barrier()

    def body(j, carry):
        pltpu.sync_copy(ones_v, deg_sh.at[idx_v.at[j]], add=True)
        return carry

    lax.fori_loop(0, CH, body, 0)
    plsc.subcore_barrier()

    @pl.when(sid == 0)
    def _():
        pltpu.sync_copy(deg_sh, out_hbm.at[cid])


@functools.partial(
    pl.kernel,
    out_type=jax.ShapeDtypeStruct((2, NPAD, D), jnp.float32),
    mesh=plsc.VectorSubcoreMesh(**_MESH),
    scratch_types=[
        pltpu.VMEM((CH, K), jnp.int32),
        pltpu.VMEM((CH, K), jnp.int32),
        pltpu.VMEM((2, K, D), jnp.float32),
        pltpu.VMEM_SHARED((NPAD, D), jnp.float32),
        pltpu.SemaphoreType.DMA((2,)),
    ],
)
def _sc_aggregate(g_hbm, srcw_hbm, dstw_hbm, zeros_hbm, out_hbm,
                  sidx, didx, bufs, agg_sh, sems):
    cid = lax.axis_index("c")
    sid = lax.axis_index("s")
    wid = cid * NS + sid
    pltpu.sync_copy(srcw_hbm.at[wid], sidx)
    pltpu.sync_copy(dstw_hbm.at[wid], didx)

    @pl.when(sid == 0)
    def _():
        pltpu.sync_copy(zeros_hbm, agg_sh)

    plsc.subcore_barrier()

    pltpu.async_copy(g_hbm.at[sidx.at[0]], bufs.at[0], sems.at[0])

    def body(j, carry):
        b = lax.rem(j, 2)
        nb = lax.rem(j + 1, 2)

        @pl.when(j + 1 < CH)
        def _():
            pltpu.async_copy(g_hbm.at[sidx.at[j + 1]], bufs.at[nb], sems.at[nb])

        pltpu.make_async_copy(g_hbm.at[sidx.at[j]], bufs.at[b], sems.at[b]).wait()
        pltpu.sync_copy(bufs.at[b], agg_sh.at[didx.at[j]], add=True)
        return carry

    lax.fori_loop(0, CH, body, 0)
    plsc.subcore_barrier()

    @pl.when(sid == 0)
    def _():
        pltpu.sync_copy(agg_sh, out_hbm.at[cid])


@functools.partial(
    pl.kernel,
    out_type=jax.ShapeDtypeStruct((EPAD, D), jnp.float32),
    mesh=plsc.VectorSubcoreMesh(**_MESH),
    scratch_types=[
        pltpu.VMEM((CH, K), jnp.int32),
        pltpu.VMEM((CH, K), jnp.int32),
        pltpu.VMEM((2, K, D), jnp.float32),
        pltpu.VMEM((2, K, D), jnp.float32),
        pltpu.SemaphoreType.DMA((2,)),
        pltpu.SemaphoreType.DMA((2,)),
    ],
)
def _sc_edge_sum(zp_hbm, srcw_hbm, dstw_hbm, out_hbm,
                 sidx, didx, abufs, bbufs, asems, bsems):
    cid = lax.axis_index("c")
    sid = lax.axis_index("s")
    wid = cid * NS + sid
    pltpu.sync_copy(srcw_hbm.at[wid], sidx)
    pltpu.sync_copy(dstw_hbm.at[wid], didx)

    pltpu.async_copy(zp_hbm.at[sidx.at[0]], abufs.at[0], asems.at[0])
    pltpu.async_copy(zp_hbm.at[didx.at[0]], bbufs.at[0], bsems.at[0])

    def body(j, carry):
        b = lax.rem(j, 2)
        nb = lax.rem(j + 1, 2)

        @pl.when(j + 1 < CH)
        def _():
            pltpu.async_copy(zp_hbm.at[sidx.at[j + 1]], abufs.at[nb], asems.at[nb])
            pltpu.async_copy(zp_hbm.at[didx.at[j + 1]], bbufs.at[nb], bsems.at[nb])

        pltpu.make_async_copy(zp_hbm.at[sidx.at[j]], abufs.at[b], asems.at[b]).wait()
        pltpu.make_async_copy(zp_hbm.at[didx.at[j]], bbufs.at[b], bsems.at[b]).wait()

        ab = abufs.at[b]
        bb = bbufs.at[b]

        def row(r, c2):
            for c in range(D // 16):
                sl = pl.ds(c * 16, 16)
                ab[r, sl] = ab[r, sl] + bb[r, sl]
            return c2

        lax.fori_loop(0, K, row, 0)
        pltpu.sync_copy(ab, out_hbm.at[pl.ds(wid * EPW + j * K, K)])
        return carry

    lax.fori_loop(0, CH, body, 0)


_BLK = 1024


def _tc_prologue_body(x_ref, w_ref, degp_ref, g_ref, dinv_ref):
    h = jnp.dot(x_ref[...], w_ref[...], preferred_element_type=jnp.float32)
    degsum = degp_ref[0, :, 0:1] + degp_ref[1, :, 0:1] + 1.0
    dinv = lax.rsqrt(degsum)
    g_ref[...] = h * dinv
    dinv_ref[...] = dinv


def _tc_prologue(x, W_gcn, degp):
    grid = (N + _BLK - 1) // _BLK
    return pl.pallas_call(
        _tc_prologue_body,
        grid=(grid,),
        in_specs=[
            pl.BlockSpec((_BLK, D), lambda i: (i, 0)),
            pl.BlockSpec((D, D), lambda i: (0, 0)),
            pl.BlockSpec((2, _BLK, 8), lambda i: (0, i, 0)),
        ],
        out_specs=[
            pl.BlockSpec((_BLK, D), lambda i: (i, 0)),
            pl.BlockSpec((_BLK, 1), lambda i: (i, 0)),
        ],
        out_shape=[
            jax.ShapeDtypeStruct((N, D), jnp.float32),
            jax.ShapeDtypeStruct((N, 1), jnp.float32),
        ],
    )(x, W_gcn, degp)


def _tc_node_mlp_body(agg_ref, g_ref, dinv_ref, bg_ref, w1_ref, b1_ref,
                      w2_ref, b2_ref, z_ref):
    a = agg_ref[0] + agg_ref[1] + g_ref[...]
    x1 = jnp.maximum(a * dinv_ref[...] + bg_ref[...], 0.0)
    x2 = jnp.maximum(
        jnp.dot(x1, w1_ref[...], preferred_element_type=jnp.float32) + b1_ref[...], 0.0)
    x3 = jnp.maximum(
        jnp.dot(x2, w2_ref[...], preferred_element_type=jnp.float32) + b2_ref[...], 0.0)
    z_ref[...] = 0.5 * (
        jnp.dot(x3, w1_ref[...], preferred_element_type=jnp.float32) + b1_ref[...])


def _tc_node_mlp(agg, g, dinvc, b_gcn, W_fc1, b_fc1, W_fc2, b_fc2):
    grid = (N + _BLK - 1) // _BLK
    return pl.pallas_call(
        _tc_node_mlp_body,
        grid=(grid,),
        in_specs=[
            pl.BlockSpec((2, _BLK, D), lambda i: (0, i, 0)),
            pl.BlockSpec((_BLK, D), lambda i: (i, 0)),
            pl.BlockSpec((_BLK, 1), lambda i: (i, 0)),
            pl.BlockSpec((1, D), lambda i: (0, 0)),
            pl.BlockSpec((D, D), lambda i: (0, 0)),
            pl.BlockSpec((1, D), lambda i: (0, 0)),
            pl.BlockSpec((D, D), lambda i: (0, 0)),
            pl.BlockSpec((1, D), lambda i: (0, 0)),
        ],
        out_specs=pl.BlockSpec((_BLK, D), lambda i: (i, 0)),
        out_shape=jax.ShapeDtypeStruct((N, D), jnp.float32),
    )(agg, g, dinvc, b_gcn, W_fc1, b_fc1, W_fc2, b_fc2)


def _tc_edge_head_body(s_ref, w2_ref, b2_ref, wo_ref, bo_ref, o_ref):
    e1 = jnp.maximum(s_ref[...], 0.0)
    e2 = jnp.maximum(
        jnp.dot(e1, w2_ref[...], preferred_element_type=jnp.float32) + b2_ref[...], 0.0)
    sc = jnp.dot(e2, wo_ref[...], preferred_element_type=jnp.float32) + bo_ref[...]
    s0 = sc[:, 0:1]
    s1 = sc[:, 1:2]
    m = jnp.maximum(s0, s1)
    lse = m + jnp.log(jnp.exp(s0 - m) + jnp.exp(s1 - m))
    o_ref[...] = sc - lse


def _tc_edge_head(s, W_fc2, b_fc2, W_out, b_out):
    grid = EPAD // _BLK
    return pl.pallas_call(
        _tc_edge_head_body,
        grid=(grid,),
        in_specs=[
            pl.BlockSpec((_BLK, D), lambda i: (i, 0)),
            pl.BlockSpec((D, D), lambda i: (0, 0)),
            pl.BlockSpec((1, D), lambda i: (0, 0)),
            pl.BlockSpec((D, 2), lambda i: (0, 0)),
            pl.BlockSpec((1, 2), lambda i: (0, 0)),
        ],
        out_specs=pl.BlockSpec((_BLK, 2), lambda i: (i, 0)),
        out_shape=jax.ShapeDtypeStruct((EPAD, 2), jnp.float32),
    )(s, W_fc2, b_fc2, W_out, b_out)


def kernel(x, edge_index, W_gcn, b_gcn, W_fc1, b_fc1, W_fc2, b_fc2, W_out, b_out):
    src = edge_index[0].astype(jnp.int32)
    dst = edge_index[1].astype(jnp.int32)
    pad = EPAD - E
    srcw = jnp.concatenate([src, jnp.zeros((pad,), jnp.int32)]).reshape(NW, CH, K)
    dstw = jnp.concatenate([dst, jnp.full((pad,), N, jnp.int32)]).reshape(NW, CH, K)
    ones8 = jnp.ones((K, 8), jnp.float32)
    zeros8 = jnp.zeros((NPAD, 8), jnp.float32)
    zerosD = jnp.zeros((NPAD, D), jnp.float32)

    degp = _sc_degree(dstw, ones8, zeros8)
    g, dinvc = _tc_prologue(x, W_gcn, degp)
    agg = _sc_aggregate(g, srcw, dstw, zerosD)
    z = _tc_node_mlp(agg, g, dinvc, b_gcn.reshape(1, D), W_fc1,
                     b_fc1.reshape(1, D), W_fc2, b_fc2.reshape(1, D))
    s = _sc_edge_sum(z, srcw, dstw)
    out = _tc_edge_head(s, W_fc2, b_fc2.reshape(1, D), W_out, b_out.reshape(1, 2))
    return out[:E]


# speedup vs baseline: 5.7527x; 5.7527x over previous
import functools

import jax
import jax.numpy as jnp
from jax import lax
from jax.experimental import pallas as pl
from jax.experimental.pallas import tpu as pltpu
from jax.experimental.pallas import tpu_sc as plsc

N = 10000
E = 320000
D = 128
NPAD = 10016
NW = 32
NS = 16
K = 128
CH = 79
EPW = CH * K
EPAD = NW * EPW

_MESH = dict(core_axis_name="c", subcore_axis_name="s")


@functools.partial(
    pl.kernel,
    out_type=jax.ShapeDtypeStruct((2, NPAD, D), jnp.float32),
    mesh=plsc.VectorSubcoreMesh(**_MESH),
    scratch_types=[
        pltpu.VMEM((CH, K), jnp.int32),
        pltpu.VMEM((K, D), jnp.float32),
        pltpu.VMEM_SHARED((NPAD, D), jnp.float32),
    ],
)
def _sc_degree(dstw_hbm, ones_hbm, zeros_hbm, out_hbm, idx_v, ones_v, deg_sh):
    cid = lax.axis_index("c")
    sid = lax.axis_index("s")
    wid = cid * NS + sid
    pltpu.sync_copy(dstw_hbm.at[wid], idx_v)
    pltpu.sync_copy(ones_hbm, ones_v)

    @pl.when(sid == 0)
    def _():
        pltpu.sync_copy(zeros_hbm, deg_sh)

    plsc.subcore_barrier()

    def body(j, carry):
        pltpu.sync_copy(ones_v, deg_sh.at[idx_v.at[j]], add=True)
        return carry

    lax.fori_loop(0, CH, body, 0)
    plsc.subcore_barrier()

    @pl.when(sid == 0)
    def _():
        pltpu.sync_copy(deg_sh, out_hbm.at[cid])


@functools.partial(
    pl.kernel,
    out_type=jax.ShapeDtypeStruct((2, NPAD, D), jnp.float32),
    mesh=plsc.VectorSubcoreMesh(**_MESH),
    scratch_types=[
        pltpu.VMEM((CH, K), jnp.int32),
        pltpu.VMEM((CH, K), jnp.int32),
        pltpu.VMEM((K, D), jnp.float32),
        pltpu.VMEM_SHARED((NPAD, D), jnp.float32),
        pltpu.SemaphoreType.DMA,
    ],
)
def _sc_aggregate(g_hbm, srcw_hbm, dstw_hbm, zeros_hbm, out_hbm,
                  sidx, didx, buf, agg_sh, sem):
    cid = lax.axis_index("c")
    sid = lax.axis_index("s")
    wid = cid * NS + sid
    pltpu.sync_copy(srcw_hbm.at[wid], sidx)
    pltpu.sync_copy(dstw_hbm.at[wid], didx)

    @pl.when(sid == 0)
    def _():
        pltpu.sync_copy(zeros_hbm, agg_sh)

    plsc.subcore_barrier()

    def body(j, carry):
        pltpu.async_copy(g_hbm.at[sidx.at[j]], buf, sem).wait()
        pltpu.sync_copy(buf, agg_sh.at[didx.at[j]], add=True)
        return carry

    lax.fori_loop(0, CH, body, 0)
    plsc.subcore_barrier()

    @pl.when(sid == 0)
    def _():
        pltpu.sync_copy(agg_sh, out_hbm.at[cid])


@functools.partial(
    pl.kernel,
    out_type=jax.ShapeDtypeStruct((EPAD, D), jnp.float32),
    mesh=plsc.VectorSubcoreMesh(**_MESH),
    scratch_types=[
        pltpu.VMEM((CH, K), jnp.int32),
        pltpu.VMEM((CH, K), jnp.int32),
        pltpu.VMEM((2, K, D), jnp.float32),
        pltpu.VMEM((2, K, D), jnp.float32),
        pltpu.SemaphoreType.DMA((2,)),
        pltpu.SemaphoreType.DMA((2,)),
    ],
)
def _sc_edge_sum(zp_hbm, srcw_hbm, dstw_hbm, out_hbm,
                 sidx, didx, abufs, bbufs, asems, bsems):
    cid = lax.axis_index("c")
    sid = lax.axis_index("s")
    wid = cid * NS + sid
    pltpu.sync_copy(srcw_hbm.at[wid], sidx)
    pltpu.sync_copy(dstw_hbm.at[wid], didx)

    pltpu.async_copy(zp_hbm.at[sidx.at[0]], abufs.at[0], asems.at[0])
    pltpu.async_copy(zp_hbm.at[didx.at[0]], bbufs.at[0], bsems.at[0])

    def body(j, carry):
        b = lax.rem(j, 2)
        nb = lax.rem(j + 1, 2)

        @pl.when(j + 1 < CH)
        def _():
            pltpu.async_copy(zp_hbm.at[sidx.at[j + 1]], abufs.at[nb], asems.at[nb])
            pltpu.async_copy(zp_hbm.at[didx.at[j + 1]], bbufs.at[nb], bsems.at[nb])

        pltpu.make_async_copy(zp_hbm.at[sidx.at[j]], abufs.at[b], asems.at[b]).wait()
        pltpu.make_async_copy(zp_hbm.at[didx.at[j]], bbufs.at[b], bsems.at[b]).wait()

        ab = abufs.at[b]
        bb = bbufs.at[b]

        def row(r, c2):
            for c in range(D // 16):
                sl = pl.ds(c * 16, 16)
                ab[r, sl] = ab[r, sl] + bb[r, sl]
            return c2

        lax.fori_loop(0, K, row, 0)
        pltpu.sync_copy(ab, out_hbm.at[pl.ds(wid * EPW + j * K, K)])
        return carry

    lax.fori_loop(0, CH, body, 0)


_BLK = 1024


def _tc_prologue_body(x_ref, w_ref, degp_ref, g_ref, dinv_ref):
    h = jnp.dot(x_ref[...], w_ref[...], preferred_element_type=jnp.float32)
    degsum = degp_ref[0, :, 0:1] + degp_ref[1, :, 0:1] + 1.0
    dinv = lax.rsqrt(degsum)
    g_ref[...] = h * dinv
    dinv_ref[...] = dinv


def _tc_prologue(x, W_gcn, degp):
    grid = (N + _BLK - 1) // _BLK
    return pl.pallas_call(
        _tc_prologue_body,
        grid=(grid,),
        in_specs=[
            pl.BlockSpec((_BLK, D), lambda i: (i, 0)),
            pl.BlockSpec((D, D), lambda i: (0, 0)),
            pl.BlockSpec((2, _BLK, D), lambda i: (0, i, 0)),
        ],
        out_specs=[
            pl.BlockSpec((_BLK, D), lambda i: (i, 0)),
            pl.BlockSpec((_BLK, 1), lambda i: (i, 0)),
        ],
        out_shape=[
            jax.ShapeDtypeStruct((N, D), jnp.float32),
            jax.ShapeDtypeStruct((N, 1), jnp.float32),
        ],
    )(x, W_gcn, degp)


def _tc_node_mlp_body(agg_ref, g_ref, dinv_ref, bg_ref, w1_ref, b1_ref,
                      w2_ref, b2_ref, z_ref):
    a = agg_ref[0] + agg_ref[1] + g_ref[...]
    x1 = jnp.maximum(a * dinv_ref[...] + bg_ref[...], 0.0)
    x2 = jnp.maximum(
        jnp.dot(x1, w1_ref[...], preferred_element_type=jnp.float32) + b1_ref[...], 0.0)
    x3 = jnp.maximum(
        jnp.dot(x2, w2_ref[...], preferred_element_type=jnp.float32) + b2_ref[...], 0.0)
    z_ref[...] = 0.5 * (
        jnp.dot(x3, w1_ref[...], preferred_element_type=jnp.float32) + b1_ref[...])


def _tc_node_mlp(agg, g, dinvc, b_gcn, W_fc1, b_fc1, W_fc2, b_fc2):
    grid = (N + _BLK - 1) // _BLK
    return pl.pallas_call(
        _tc_node_mlp_body,
        grid=(grid,),
        in_specs=[
            pl.BlockSpec((2, _BLK, D), lambda i: (0, i, 0)),
            pl.BlockSpec((_BLK, D), lambda i: (i, 0)),
            pl.BlockSpec((_BLK, 1), lambda i: (i, 0)),
            pl.BlockSpec((1, D), lambda i: (0, 0)),
            pl.BlockSpec((D, D), lambda i: (0, 0)),
            pl.BlockSpec((1, D), lambda i: (0, 0)),
            pl.BlockSpec((D, D), lambda i: (0, 0)),
            pl.BlockSpec((1, D), lambda i: (0, 0)),
        ],
        out_specs=pl.BlockSpec((_BLK, D), lambda i: (i, 0)),
        out_shape=jax.ShapeDtypeStruct((N, D), jnp.float32),
    )(agg, g, dinvc, b_gcn, W_fc1, b_fc1, W_fc2, b_fc2)


def _tc_edge_head_body(s_ref, w2_ref, b2_ref, wo_ref, bo_ref, o_ref):
    e1 = jnp.maximum(s_ref[...], 0.0)
    e2 = jnp.maximum(
        jnp.dot(e1, w2_ref[...], preferred_element_type=jnp.float32) + b2_ref[...], 0.0)
    sc = jnp.dot(e2, wo_ref[...], preferred_element_type=jnp.float32) + bo_ref[...]
    s0 = sc[:, 0:1]
    s1 = sc[:, 1:2]
    m = jnp.maximum(s0, s1)
    lse = m + jnp.log(jnp.exp(s0 - m) + jnp.exp(s1 - m))
    o_ref[...] = sc - lse


def _tc_edge_head(s, W_fc2, b_fc2, W_out, b_out):
    grid = EPAD // _BLK
    return pl.pallas_call(
        _tc_edge_head_body,
        grid=(grid,),
        in_specs=[
            pl.BlockSpec((_BLK, D), lambda i: (i, 0)),
            pl.BlockSpec((D, D), lambda i: (0, 0)),
            pl.BlockSpec((1, D), lambda i: (0, 0)),
            pl.BlockSpec((D, 2), lambda i: (0, 0)),
            pl.BlockSpec((1, 2), lambda i: (0, 0)),
        ],
        out_specs=pl.BlockSpec((_BLK, 2), lambda i: (i, 0)),
        out_shape=jax.ShapeDtypeStruct((EPAD, 2), jnp.float32),
    )(s, W_fc2, b_fc2, W_out, b_out)


def kernel(x, edge_index, W_gcn, b_gcn, W_fc1, b_fc1, W_fc2, b_fc2, W_out, b_out):
    src = edge_index[0].astype(jnp.int32)
    dst = edge_index[1].astype(jnp.int32)
    pad = EPAD - E
    srcw = jnp.concatenate([src, jnp.zeros((pad,), jnp.int32)]).reshape(NW, CH, K)
    dstw = jnp.concatenate([dst, jnp.full((pad,), N, jnp.int32)]).reshape(NW, CH, K)
    onesD = jnp.ones((K, D), jnp.float32)
    zerosD = jnp.zeros((NPAD, D), jnp.float32)

    degp = _sc_degree(dstw, onesD, zerosD)
    g, dinvc = _tc_prologue(x, W_gcn, degp)
    agg = _sc_aggregate(g, srcw, dstw, zerosD)
    z = _tc_node_mlp(agg, g, dinvc, b_gcn.reshape(1, D), W_fc1,
                     b_fc1.reshape(1, D), W_fc2, b_fc2.reshape(1, D))
    s = _sc_edge_sum(z, srcw, dstw)
    out = _tc_edge_head(s, W_fc2, b_fc2.reshape(1, D), W_out, b_out.reshape(1, 2))
    return out[:E]


# speedup vs baseline: 6.0067x; 1.0441x over previous
import functools

import jax
import jax.numpy as jnp
from jax import lax
from jax.experimental import pallas as pl
from jax.experimental.pallas import tpu as pltpu
from jax.experimental.pallas import tpu_sc as plsc

N = 10000
E = 320000
D = 128
NPAD = 10008
NW = 32
NS = 16
K = 128
CH = 79
EPW = CH * K
EPAD = NW * EPW

_MESH = dict(core_axis_name="c", subcore_axis_name="s")


@functools.partial(
    pl.kernel,
    out_type=jax.ShapeDtypeStruct((2, NPAD, D), jnp.float32),
    mesh=plsc.VectorSubcoreMesh(**_MESH),
    scratch_types=[
        pltpu.VMEM((CH, K), jnp.int32),
        pltpu.VMEM((K, D), jnp.float32),
        pltpu.VMEM_SHARED((NPAD, D), jnp.float32),
        pltpu.SemaphoreType.DMA((2,)),
    ],
)
def _sc_degree(dstw_hbm, ones_hbm, zeros_hbm, out_hbm, idx_v, ones_v, deg_sh, sems):
    cid = lax.axis_index("c")
    sid = lax.axis_index("s")
    wid = cid * NS + sid
    pltpu.sync_copy(dstw_hbm.at[wid], idx_v)
    pltpu.sync_copy(ones_hbm, ones_v)

    @pl.when(sid == 0)
    def _():
        pltpu.sync_copy(zeros_hbm, deg_sh)

    plsc.subcore_barrier()

    def body(j, carry):
        b = lax.rem(j, 2)

        @pl.when(j >= 2)
        def _():
            pltpu.make_async_copy(ones_v, deg_sh.at[idx_v.at[j]], sems.at[b]).wait()

        pltpu.async_copy(ones_v, deg_sh.at[idx_v.at[j]], sems.at[b], add=True)
        return carry

    lax.fori_loop(0, CH, body, 0)
    pltpu.make_async_copy(ones_v, deg_sh.at[idx_v.at[0]], sems.at[0]).wait()
    pltpu.make_async_copy(ones_v, deg_sh.at[idx_v.at[0]], sems.at[1]).wait()
    plsc.subcore_barrier()

    @pl.when(sid == 0)
    def _():
        pltpu.sync_copy(deg_sh, out_hbm.at[cid])


@functools.partial(
    pl.kernel,
    out_type=jax.ShapeDtypeStruct((2, NPAD, D), jnp.float32),
    mesh=plsc.VectorSubcoreMesh(**_MESH),
    scratch_types=[
        pltpu.VMEM((2, K), jnp.int32),
        pltpu.VMEM((CH, K), jnp.int32),
        pltpu.VMEM((2, K, D), jnp.float32),
        pltpu.VMEM_SHARED((NPAD, D), jnp.float32),
        pltpu.SemaphoreType.DMA((2,)),
        pltpu.SemaphoreType.DMA((2,)),
        pltpu.SemaphoreType.DMA((2,)),
    ],
)
def _sc_aggregate(g_hbm, srcw_hbm, dstw_hbm, zeros_hbm, out_hbm,
                  sidxb, didx, bufs, agg_sh, semi, semg, sems):
    cid = lax.axis_index("c")
    sid = lax.axis_index("s")
    wid = cid * NS + sid
    pltpu.sync_copy(dstw_hbm.at[wid], didx)
    pltpu.sync_copy(srcw_hbm.at[wid, 0], sidxb.at[0])

    @pl.when(sid == 0)
    def _():
        pltpu.sync_copy(zeros_hbm, agg_sh)

    plsc.subcore_barrier()

    pltpu.async_copy(srcw_hbm.at[wid, 1], sidxb.at[1], semi.at[1])
    pltpu.async_copy(g_hbm.at[sidxb.at[0]], bufs.at[0], semg.at[0])

    def body(j, carry):
        b = lax.rem(j, 2)
        nb = lax.rem(j + 1, 2)

        @pl.when(j >= 1)
        def _():
            pltpu.make_async_copy(bufs.at[nb], agg_sh.at[didx.at[j]],
                                  sems.at[nb]).wait()

        @pl.when(j + 1 < CH)
        def _():
            pltpu.make_async_copy(srcw_hbm.at[wid, j + 1], sidxb.at[nb],
                                  semi.at[nb]).wait()
            pltpu.async_copy(g_hbm.at[sidxb.at[nb]], bufs.at[nb], semg.at[nb])

        pltpu.make_async_copy(g_hbm.at[sidxb.at[b]], bufs.at[b], semg.at[b]).wait()

        @pl.when(j + 2 < CH)
        def _():
            pltpu.async_copy(srcw_hbm.at[wid, j + 2], sidxb.at[b], semi.at[b])

        pltpu.async_copy(bufs.at[b], agg_sh.at[didx.at[j]], sems.at[b], add=True)
        return carry

    lax.fori_loop(0, CH, body, 0)
    pltpu.make_async_copy(bufs.at[0], agg_sh.at[didx.at[0]],
                          sems.at[lax.rem(CH - 1, 2)]).wait()
    plsc.subcore_barrier()

    @pl.when(sid == 0)
    def _():
        pltpu.sync_copy(agg_sh, out_hbm.at[cid])


@functools.partial(
    pl.kernel,
    out_type=jax.ShapeDtypeStruct((EPAD, D), jnp.float32),
    mesh=plsc.VectorSubcoreMesh(**_MESH),
    scratch_types=[
        pltpu.VMEM((CH, K), jnp.int32),
        pltpu.VMEM((CH, K), jnp.int32),
        pltpu.VMEM((2, K, D), jnp.float32),
        pltpu.VMEM((2, K, D), jnp.float32),
        pltpu.SemaphoreType.DMA((2,)),
        pltpu.SemaphoreType.DMA((2,)),
        pltpu.SemaphoreType.DMA((2,)),
    ],
)
def _sc_edge_sum(zp_hbm, srcw_hbm, dstw_hbm, out_hbm,
                 sidx, didx, abufs, bbufs, asems, bsems, osems):
    cid = lax.axis_index("c")
    sid = lax.axis_index("s")
    wid = cid * NS + sid
    pltpu.sync_copy(srcw_hbm.at[wid], sidx)
    pltpu.sync_copy(dstw_hbm.at[wid], didx)

    pltpu.async_copy(zp_hbm.at[sidx.at[0]], abufs.at[0], asems.at[0])
    pltpu.async_copy(zp_hbm.at[didx.at[0]], bbufs.at[0], bsems.at[0])

    def out_slice(j):
        return out_hbm.at[pl.ds(wid * EPW + j * K, K)]

    def body(j, carry):
        b = lax.rem(j, 2)
        nb = lax.rem(j + 1, 2)

        @pl.when(j >= 1)
        def _():
            pltpu.make_async_copy(abufs.at[nb], out_slice(j), osems.at[nb]).wait()

        @pl.when(j + 1 < CH)
        def _():
            pltpu.async_copy(zp_hbm.at[sidx.at[j + 1]], abufs.at[nb], asems.at[nb])
            pltpu.async_copy(zp_hbm.at[didx.at[j + 1]], bbufs.at[nb], bsems.at[nb])

        pltpu.make_async_copy(zp_hbm.at[sidx.at[j]], abufs.at[b], asems.at[b]).wait()
        pltpu.make_async_copy(zp_hbm.at[didx.at[j]], bbufs.at[b], bsems.at[b]).wait()

        ab = abufs.at[b]
        bb = bbufs.at[b]

        def row(r, c2):
            for c in range(D // 16):
                sl = pl.ds(c * 16, 16)
                ab[r, sl] = ab[r, sl] + bb[r, sl]
            return c2

        lax.fori_loop(0, K, row, 0)
        pltpu.async_copy(ab, out_slice(j), osems.at[b])
        return carry

    lax.fori_loop(0, CH, body, 0)
    pltpu.make_async_copy(abufs.at[0], out_slice(0), osems.at[lax.rem(CH - 1, 2)]).wait()


_BLK = 1024


def _tc_prologue_body(x_ref, w_ref, degp_ref, g_ref, dinv_ref):
    h = jnp.dot(x_ref[...], w_ref[...], preferred_element_type=jnp.float32)
    degsum = degp_ref[0, :, 0:1] + degp_ref[1, :, 0:1] + 1.0
    dinv = lax.rsqrt(degsum)
    g_ref[...] = h * dinv
    dinv_ref[...] = dinv


def _tc_prologue(x, W_gcn, degp):
    grid = (N + _BLK - 1) // _BLK
    return pl.pallas_call(
        _tc_prologue_body,
        grid=(grid,),
        in_specs=[
            pl.BlockSpec((_BLK, D), lambda i: (i, 0)),
            pl.BlockSpec((D, D), lambda i: (0, 0)),
            pl.BlockSpec((2, _BLK, D), lambda i: (0, i, 0)),
        ],
        out_specs=[
            pl.BlockSpec((_BLK, D), lambda i: (i, 0)),
            pl.BlockSpec((_BLK, 1), lambda i: (i, 0)),
        ],
        out_shape=[
            jax.ShapeDtypeStruct((N, D), jnp.float32),
            jax.ShapeDtypeStruct((N, 1), jnp.float32),
        ],
    )(x, W_gcn, degp)


def _tc_node_mlp_body(agg_ref, g_ref, dinv_ref, bg_ref, w1_ref, b1_ref,
                      w2_ref, b2_ref, z_ref):
    a = agg_ref[0] + agg_ref[1] + g_ref[...]
    x1 = jnp.maximum(a * dinv_ref[...] + bg_ref[...], 0.0)
    x2 = jnp.maximum(
        jnp.dot(x1, w1_ref[...], preferred_element_type=jnp.float32) + b1_ref[...], 0.0)
    x3 = jnp.maximum(
        jnp.dot(x2, w2_ref[...], preferred_element_type=jnp.float32) + b2_ref[...], 0.0)
    z_ref[...] = 0.5 * (
        jnp.dot(x3, w1_ref[...], preferred_element_type=jnp.float32) + b1_ref[...])


def _tc_node_mlp(agg, g, dinvc, b_gcn, W_fc1, b_fc1, W_fc2, b_fc2):
    grid = (NPAD + _BLK - 1) // _BLK
    return pl.pallas_call(
        _tc_node_mlp_body,
        grid=(grid,),
        in_specs=[
            pl.BlockSpec((2, _BLK, D), lambda i: (0, i, 0)),
            pl.BlockSpec((_BLK, D), lambda i: (i, 0)),
            pl.BlockSpec((_BLK, 1), lambda i: (i, 0)),
            pl.BlockSpec((1, D), lambda i: (0, 0)),
            pl.BlockSpec((D, D), lambda i: (0, 0)),
            pl.BlockSpec((1, D), lambda i: (0, 0)),
            pl.BlockSpec((D, D), lambda i: (0, 0)),
            pl.BlockSpec((1, D), lambda i: (0, 0)),
        ],
        out_specs=pl.BlockSpec((_BLK, D), lambda i: (i, 0)),
        out_shape=jax.ShapeDtypeStruct((NPAD, D), jnp.float32),
    )(agg, g, dinvc, b_gcn, W_fc1, b_fc1, W_fc2, b_fc2)


def _tc_edge_head_body(s_ref, w2_ref, b2_ref, wo_ref, bo_ref, o_ref):
    e1 = jnp.maximum(s_ref[...], 0.0)
    e2 = jnp.maximum(
        jnp.dot(e1, w2_ref[...], preferred_element_type=jnp.float32) + b2_ref[...], 0.0)
    sc = jnp.dot(e2, wo_ref[...], preferred_element_type=jnp.float32) + bo_ref[...]
    s0 = sc[:, 0:1]
    s1 = sc[:, 1:2]
    m = jnp.maximum(s0, s1)
    lse = m + jnp.log(jnp.exp(s0 - m) + jnp.exp(s1 - m))
    o_ref[...] = sc - lse


def _tc_edge_head(s, W_fc2, b_fc2, W_out, b_out):
    grid = EPAD // _BLK
    return pl.pallas_call(
        _tc_edge_head_body,
        grid=(grid,),
        in_specs=[
            pl.BlockSpec((_BLK, D), lambda i: (i, 0)),
            pl.BlockSpec((D, D), lambda i: (0, 0)),
            pl.BlockSpec((1, D), lambda i: (0, 0)),
            pl.BlockSpec((D, 2), lambda i: (0, 0)),
            pl.BlockSpec((1, 2), lambda i: (0, 0)),
        ],
        out_specs=pl.BlockSpec((_BLK, 2), lambda i: (i, 0)),
        out_shape=jax.ShapeDtypeStruct((EPAD, 2), jnp.float32),
    )(s, W_fc2, b_fc2, W_out, b_out)


def kernel(x, edge_index, W_gcn, b_gcn, W_fc1, b_fc1, W_fc2, b_fc2, W_out, b_out):
    src = edge_index[0].astype(jnp.int32)
    dst = edge_index[1].astype(jnp.int32)
    pad = EPAD - E
    srcw = jnp.concatenate([src, jnp.zeros((pad,), jnp.int32)]).reshape(NW, CH, K)
    dstw = jnp.concatenate([dst, jnp.full((pad,), N, jnp.int32)]).reshape(NW, CH, K)
    onesD = jnp.ones((K, D), jnp.float32)
    zerosD = jnp.zeros((NPAD, D), jnp.float32)

    degp = _sc_degree(dstw, onesD, zerosD)
    g, dinvc = _tc_prologue(x, W_gcn, degp)
    agg = _sc_aggregate(g, srcw, dstw, zerosD)
    z = _tc_node_mlp(agg, g, dinvc, b_gcn.reshape(1, D), W_fc1,
                     b_fc1.reshape(1, D), W_fc2, b_fc2.reshape(1, D))
    s = _sc_edge_sum(z, srcw, dstw)
    out = _tc_edge_head(s, W_fc2, b_fc2.reshape(1, D), W_out, b_out.reshape(1, 2))
    return out[:E]


# speedup vs baseline: 6.1248x; 1.0197x over previous
import functools

import jax
import jax.numpy as jnp
from jax import lax
from jax.experimental import pallas as pl
from jax.experimental.pallas import tpu as pltpu
from jax.experimental.pallas import tpu_sc as plsc

N = 10000
E = 320000
D = 128
NPAD = 10008
NW = 32
NS = 16
K = 128
CH = 79
EPW = CH * K
EPAD = NW * EPW

_MESH = dict(core_axis_name="c", subcore_axis_name="s")


@functools.partial(
    pl.kernel,
    out_type=jax.ShapeDtypeStruct((2, NPAD, D), jnp.float32),
    mesh=plsc.VectorSubcoreMesh(**_MESH),
    scratch_types=[
        pltpu.VMEM((CH, K), jnp.int32),
        pltpu.VMEM((K, D), jnp.float32),
        pltpu.VMEM_SHARED((NPAD, D), jnp.float32),
        pltpu.SemaphoreType.DMA((2,)),
    ],
)
def _sc_degree(dstw_hbm, ones_hbm, zeros_hbm, out_hbm, idx_v, ones_v, deg_sh, sems):
    cid = lax.axis_index("c")
    sid = lax.axis_index("s")
    wid = cid * NS + sid
    pltpu.sync_copy(dstw_hbm.at[wid], idx_v)
    pltpu.sync_copy(ones_hbm, ones_v)

    @pl.when(sid == 0)
    def _():
        pltpu.sync_copy(zeros_hbm, deg_sh)

    plsc.subcore_barrier()

    def body(j, carry):
        b = lax.rem(j, 2)

        @pl.when(j >= 2)
        def _():
            pltpu.make_async_copy(ones_v, deg_sh.at[idx_v.at[j]], sems.at[b]).wait()

        pltpu.async_copy(ones_v, deg_sh.at[idx_v.at[j]], sems.at[b], add=True)
        return carry

    lax.fori_loop(0, CH, body, 0)
    pltpu.make_async_copy(ones_v, deg_sh.at[idx_v.at[0]], sems.at[0]).wait()
    pltpu.make_async_copy(ones_v, deg_sh.at[idx_v.at[0]], sems.at[1]).wait()
    plsc.subcore_barrier()

    @pl.when(sid == 0)
    def _():
        pltpu.sync_copy(deg_sh, out_hbm.at[cid])


@functools.partial(
    pl.kernel,
    out_type=jax.ShapeDtypeStruct((2, NPAD, D), jnp.float32),
    mesh=plsc.VectorSubcoreMesh(**_MESH),
    scratch_types=[
        pltpu.VMEM((2, K), jnp.int32),
        pltpu.VMEM((CH, K), jnp.int32),
        pltpu.VMEM((2, K, D), jnp.float32),
        pltpu.VMEM_SHARED((NPAD, D), jnp.float32),
        pltpu.SemaphoreType.DMA((2,)),
        pltpu.SemaphoreType.DMA((2,)),
        pltpu.SemaphoreType.DMA((2,)),
    ],
)
def _sc_aggregate(g_hbm, srcw_hbm, dstw_hbm, zeros_hbm, out_hbm,
                  sidxb, didx, bufs, agg_sh, semi, semg, sems):
    cid = lax.axis_index("c")
    sid = lax.axis_index("s")
    wid = cid * NS + sid
    pltpu.sync_copy(dstw_hbm.at[wid], didx)
    pltpu.sync_copy(srcw_hbm.at[wid, 0], sidxb.at[0])

    @pl.when(sid == 0)
    def _():
        pltpu.sync_copy(zeros_hbm, agg_sh)

    plsc.subcore_barrier()

    pltpu.async_copy(srcw_hbm.at[wid, 1], sidxb.at[1], semi.at[1])
    pltpu.async_copy(g_hbm.at[sidxb.at[0]], bufs.at[0], semg.at[0])

    def body(j, carry):
        b = lax.rem(j, 2)
        nb = lax.rem(j + 1, 2)

        @pl.when(j >= 1)
        def _():
            pltpu.make_async_copy(bufs.at[nb], agg_sh.at[didx.at[j]],
                                  sems.at[nb]).wait()

        @pl.when(j + 1 < CH)
        def _():
            pltpu.make_async_copy(srcw_hbm.at[wid, j + 1], sidxb.at[nb],
                                  semi.at[nb]).wait()
            pltpu.async_copy(g_hbm.at[sidxb.at[nb]], bufs.at[nb], semg.at[nb])

        pltpu.make_async_copy(g_hbm.at[sidxb.at[b]], bufs.at[b], semg.at[b]).wait()

        @pl.when(j + 2 < CH)
        def _():
            pltpu.async_copy(srcw_hbm.at[wid, j + 2], sidxb.at[b], semi.at[b])

        pltpu.async_copy(bufs.at[b], agg_sh.at[didx.at[j]], sems.at[b], add=True)
        return carry

    lax.fori_loop(0, CH, body, 0)
    pltpu.make_async_copy(bufs.at[0], agg_sh.at[didx.at[0]],
                          sems.at[lax.rem(CH - 1, 2)]).wait()
    plsc.subcore_barrier()

    @pl.when(sid == 0)
    def _():
        pltpu.sync_copy(agg_sh, out_hbm.at[cid])


@functools.partial(
    pl.kernel,
    out_type=jax.ShapeDtypeStruct((EPAD, D), jnp.float32),
    mesh=plsc.VectorSubcoreMesh(**_MESH),
    scratch_types=[
        pltpu.VMEM((CH, K), jnp.int32),
        pltpu.VMEM((CH, K), jnp.int32),
        pltpu.VMEM((3, K, D), jnp.float32),
        pltpu.VMEM((3, K, D), jnp.float32),
        pltpu.SemaphoreType.DMA((3,)),
        pltpu.SemaphoreType.DMA((3,)),
        pltpu.SemaphoreType.DMA((3,)),
    ],
)
def _sc_edge_sum(zp_hbm, srcw_hbm, dstw_hbm, out_hbm,
                 sidx, didx, abufs, bbufs, asems, bsems, osems):
    cid = lax.axis_index("c")
    sid = lax.axis_index("s")
    wid = cid * NS + sid
    pltpu.sync_copy(srcw_hbm.at[wid], sidx)
    pltpu.sync_copy(dstw_hbm.at[wid], didx)

    pltpu.async_copy(zp_hbm.at[sidx.at[0]], abufs.at[0], asems.at[0])
    pltpu.async_copy(zp_hbm.at[didx.at[0]], bbufs.at[0], bsems.at[0])

    def out_slice(j):
        return out_hbm.at[pl.ds(wid * EPW + j * K, K)]

    def body(j, carry):
        r = lax.rem(j, 3)
        r1 = lax.rem(j + 1, 3)

        @pl.when(j >= 2)
        def _():
            pltpu.make_async_copy(abufs.at[r1], out_slice(j), osems.at[r1]).wait()

        @pl.when(j + 1 < CH)
        def _():
            pltpu.async_copy(zp_hbm.at[sidx.at[j + 1]], abufs.at[r1], asems.at[r1])
            pltpu.async_copy(zp_hbm.at[didx.at[j + 1]], bbufs.at[r1], bsems.at[r1])

        pltpu.make_async_copy(zp_hbm.at[sidx.at[j]], abufs.at[r], asems.at[r]).wait()
        pltpu.make_async_copy(zp_hbm.at[didx.at[j]], bbufs.at[r], bsems.at[r]).wait()

        ab = abufs.at[r]
        bb = bbufs.at[r]

        def row(rr, c2):
            for c in range(D // 16):
                sl = pl.ds(c * 16, 16)
                ab[rr, sl] = ab[rr, sl] + bb[rr, sl]
            return c2

        lax.fori_loop(0, K, row, 0)
        pltpu.async_copy(ab, out_slice(j), osems.at[r])
        return carry

    lax.fori_loop(0, CH, body, 0)
    pltpu.make_async_copy(abufs.at[0], out_slice(0), osems.at[lax.rem(CH - 1, 3)]).wait()
    pltpu.make_async_copy(abufs.at[0], out_slice(0), osems.at[lax.rem(CH - 2, 3)]).wait()


_BLK = 1024


def _tc_prologue_body(x_ref, w_ref, degp_ref, g_ref, dinv_ref):
    h = jnp.dot(x_ref[...], w_ref[...], preferred_element_type=jnp.float32)
    degsum = degp_ref[0, :, 0:1] + degp_ref[1, :, 0:1] + 1.0
    dinv = lax.rsqrt(degsum)
    g_ref[...] = h * dinv
    dinv_ref[...] = dinv


def _tc_prologue(x, W_gcn, degp):
    grid = (N + _BLK - 1) // _BLK
    return pl.pallas_call(
        _tc_prologue_body,
        grid=(grid,),
        in_specs=[
            pl.BlockSpec((_BLK, D), lambda i: (i, 0)),
            pl.BlockSpec((D, D), lambda i: (0, 0)),
            pl.BlockSpec((2, _BLK, D), lambda i: (0, i, 0)),
        ],
        out_specs=[
            pl.BlockSpec((_BLK, D), lambda i: (i, 0)),
            pl.BlockSpec((_BLK, 1), lambda i: (i, 0)),
        ],
        out_shape=[
            jax.ShapeDtypeStruct((N, D), jnp.float32),
            jax.ShapeDtypeStruct((N, 1), jnp.float32),
        ],
    )(x, W_gcn, degp)


def _tc_node_mlp_body(agg_ref, g_ref, dinv_ref, bg_ref, w1_ref, b1_ref,
                      w2_ref, b2_ref, z_ref):
    a = agg_ref[0] + agg_ref[1] + g_ref[...]
    x1 = jnp.maximum(a * dinv_ref[...] + bg_ref[...], 0.0)
    x2 = jnp.maximum(
        jnp.dot(x1, w1_ref[...], preferred_element_type=jnp.float32) + b1_ref[...], 0.0)
    x3 = jnp.maximum(
        jnp.dot(x2, w2_ref[...], preferred_element_type=jnp.float32) + b2_ref[...], 0.0)
    z_ref[...] = 0.5 * (
        jnp.dot(x3, w1_ref[...], preferred_element_type=jnp.float32) + b1_ref[...])


def _tc_node_mlp(agg, g, dinvc, b_gcn, W_fc1, b_fc1, W_fc2, b_fc2):
    grid = (NPAD + _BLK - 1) // _BLK
    return pl.pallas_call(
        _tc_node_mlp_body,
        grid=(grid,),
        in_specs=[
            pl.BlockSpec((2, _BLK, D), lambda i: (0, i, 0)),
            pl.BlockSpec((_BLK, D), lambda i: (i, 0)),
            pl.BlockSpec((_BLK, 1), lambda i: (i, 0)),
            pl.BlockSpec((1, D), lambda i: (0, 0)),
            pl.BlockSpec((D, D), lambda i: (0, 0)),
            pl.BlockSpec((1, D), lambda i: (0, 0)),
            pl.BlockSpec((D, D), lambda i: (0, 0)),
            pl.BlockSpec((1, D), lambda i: (0, 0)),
        ],
        out_specs=pl.BlockSpec((_BLK, D), lambda i: (i, 0)),
        out_shape=jax.ShapeDtypeStruct((NPAD, D), jnp.float32),
    )(agg, g, dinvc, b_gcn, W_fc1, b_fc1, W_fc2, b_fc2)


def _tc_edge_head_body(s_ref, w2_ref, b2_ref, wo_ref, bo_ref, o_ref):
    e1 = jnp.maximum(s_ref[...], 0.0)
    e2 = jnp.maximum(
        jnp.dot(e1.astype(jnp.bfloat16), w2_ref[...].astype(jnp.bfloat16),
                preferred_element_type=jnp.float32) + b2_ref[...], 0.0)
    sc = jnp.dot(e2.astype(jnp.bfloat16), wo_ref[...].astype(jnp.bfloat16),
                 preferred_element_type=jnp.float32) + bo_ref[...]
    s0 = sc[:, 0:1]
    s1 = sc[:, 1:2]
    m = jnp.maximum(s0, s1)
    lse = m + jnp.log(jnp.exp(s0 - m) + jnp.exp(s1 - m))
    o_ref[...] = sc - lse


def _tc_edge_head(s, W_fc2, b_fc2, W_out, b_out):
    grid = EPAD // _BLK
    return pl.pallas_call(
        _tc_edge_head_body,
        grid=(grid,),
        in_specs=[
            pl.BlockSpec((_BLK, D), lambda i: (i, 0)),
            pl.BlockSpec((D, D), lambda i: (0, 0)),
            pl.BlockSpec((1, D), lambda i: (0, 0)),
            pl.BlockSpec((D, 2), lambda i: (0, 0)),
            pl.BlockSpec((1, 2), lambda i: (0, 0)),
        ],
        out_specs=pl.BlockSpec((_BLK, 2), lambda i: (i, 0)),
        out_shape=jax.ShapeDtypeStruct((EPAD, 2), jnp.float32),
    )(s, W_fc2, b_fc2, W_out, b_out)


def kernel(x, edge_index, W_gcn, b_gcn, W_fc1, b_fc1, W_fc2, b_fc2, W_out, b_out):
    src = edge_index[0].astype(jnp.int32)
    dst = edge_index[1].astype(jnp.int32)
    pad = EPAD - E
    srcw = jnp.concatenate([src, jnp.zeros((pad,), jnp.int32)]).reshape(NW, CH, K)
    dstw = jnp.concatenate([dst, jnp.full((pad,), N, jnp.int32)]).reshape(NW, CH, K)
    onesD = jnp.ones((K, D), jnp.float32)
    zerosD = jnp.zeros((NPAD, D), jnp.float32)

    degp = _sc_degree(dstw, onesD, zerosD)
    g, dinvc = _tc_prologue(x, W_gcn, degp)
    agg = _sc_aggregate(g, srcw, dstw, zerosD)
    z = _tc_node_mlp(agg, g, dinvc, b_gcn.reshape(1, D), W_fc1,
                     b_fc1.reshape(1, D), W_fc2, b_fc2.reshape(1, D))
    s = _sc_edge_sum(z, srcw, dstw)
    out = _tc_edge_head(s, W_fc2, b_fc2.reshape(1, D), W_out, b_out.reshape(1, 2))
    return out[:E]


# speedup vs baseline: 6.5384x; 1.0675x over previous
import functools

import jax
import jax.numpy as jnp
from jax import lax
from jax.experimental import pallas as pl
from jax.experimental.pallas import tpu as pltpu
from jax.experimental.pallas import tpu_sc as plsc

N = 10000
E = 320000
D = 128
NPAD = 10016
NW = 32
NS = 16
K = 128
CH = 79
EPW = CH * K
EPAD = NW * EPW

_MESH = dict(core_axis_name="c", subcore_axis_name="s")


@functools.partial(
    pl.kernel,
    out_type=jax.ShapeDtypeStruct((2, NPAD, D), jnp.float32),
    mesh=plsc.VectorSubcoreMesh(**_MESH),
    scratch_types=[
        pltpu.VMEM((CH, K), jnp.int32),
        pltpu.VMEM((K, D), jnp.float32),
        pltpu.VMEM_SHARED((NPAD, D), jnp.float32),
        pltpu.SemaphoreType.DMA((2,)),
    ],
)
def _sc_degree(dstw_hbm, ones_hbm, zeros_hbm, out_hbm, idx_v, ones_v, deg_sh, sems):
    cid = lax.axis_index("c")
    sid = lax.axis_index("s")
    wid = cid * NS + sid
    pltpu.sync_copy(dstw_hbm.at[wid], idx_v)
    pltpu.sync_copy(ones_hbm, ones_v)

    @pl.when(sid == 0)
    def _():
        pltpu.sync_copy(zeros_hbm, deg_sh)

    plsc.subcore_barrier()

    def body(j, carry):
        b = lax.rem(j, 2)

        @pl.when(j >= 2)
        def _():
            pltpu.make_async_copy(ones_v, deg_sh.at[idx_v.at[j]], sems.at[b]).wait()

        pltpu.async_copy(ones_v, deg_sh.at[idx_v.at[j]], sems.at[b], add=True)
        return carry

    lax.fori_loop(0, CH, body, 0)
    pltpu.make_async_copy(ones_v, deg_sh.at[idx_v.at[0]], sems.at[0]).wait()
    pltpu.make_async_copy(ones_v, deg_sh.at[idx_v.at[0]], sems.at[1]).wait()
    plsc.subcore_barrier()

    @pl.when(sid == 0)
    def _():
        pltpu.sync_copy(deg_sh, out_hbm.at[cid])


@functools.partial(
    pl.kernel,
    out_type=jax.ShapeDtypeStruct((2, NPAD, D), jnp.float32),
    mesh=plsc.VectorSubcoreMesh(**_MESH),
    scratch_types=[
        pltpu.VMEM((2, K), jnp.int32),
        pltpu.VMEM((CH, K), jnp.int32),
        pltpu.VMEM((2, K, D), jnp.float32),
        pltpu.VMEM_SHARED((NPAD, D), jnp.float32),
        pltpu.SemaphoreType.DMA((2,)),
        pltpu.SemaphoreType.DMA((2,)),
        pltpu.SemaphoreType.DMA((2,)),
    ],
)
def _sc_aggregate(g_hbm, srcw_hbm, dstw_hbm, zeros_hbm, out_hbm,
                  sidxb, didx, bufs, agg_sh, semi, semg, sems):
    cid = lax.axis_index("c")
    sid = lax.axis_index("s")
    wid = cid * NS + sid
    pltpu.sync_copy(dstw_hbm.at[wid], didx)
    pltpu.sync_copy(srcw_hbm.at[wid, 0], sidxb.at[0])

    @pl.when(sid == 0)
    def _():
        pltpu.sync_copy(zeros_hbm, agg_sh)

    plsc.subcore_barrier()

    pltpu.async_copy(srcw_hbm.at[wid, 1], sidxb.at[1], semi.at[1])
    pltpu.async_copy(g_hbm.at[sidxb.at[0]], bufs.at[0], semg.at[0])

    def body(j, carry):
        b = lax.rem(j, 2)
        nb = lax.rem(j + 1, 2)

        @pl.when(j >= 1)
        def _():
            pltpu.make_async_copy(bufs.at[nb], agg_sh.at[didx.at[j]],
                                  sems.at[nb]).wait()

        @pl.when(j + 1 < CH)
        def _():
            pltpu.make_async_copy(srcw_hbm.at[wid, j + 1], sidxb.at[nb],
                                  semi.at[nb]).wait()
            pltpu.async_copy(g_hbm.at[sidxb.at[nb]], bufs.at[nb], semg.at[nb])

        pltpu.make_async_copy(g_hbm.at[sidxb.at[b]], bufs.at[b], semg.at[b]).wait()

        @pl.when(j + 2 < CH)
        def _():
            pltpu.async_copy(srcw_hbm.at[wid, j + 2], sidxb.at[b], semi.at[b])

        pltpu.async_copy(bufs.at[b], agg_sh.at[didx.at[j]], sems.at[b], add=True)
        return carry

    lax.fori_loop(0, CH, body, 0)
    pltpu.make_async_copy(bufs.at[0], agg_sh.at[didx.at[0]],
                          sems.at[lax.rem(CH - 1, 2)]).wait()
    plsc.subcore_barrier()

    @pl.when(sid == 0)
    def _():
        pltpu.sync_copy(agg_sh, out_hbm.at[cid])


@functools.partial(
    pl.kernel,
    out_type=jax.ShapeDtypeStruct((EPAD, D), jnp.float32),
    mesh=plsc.VectorSubcoreMesh(**_MESH),
    scratch_types=[
        pltpu.VMEM((CH, K), jnp.int32),
        pltpu.VMEM((CH, K), jnp.int32),
        pltpu.VMEM((3, K, D), jnp.float32),
        pltpu.VMEM((3, K, D), jnp.float32),
        pltpu.SemaphoreType.DMA((3,)),
        pltpu.SemaphoreType.DMA((3,)),
        pltpu.SemaphoreType.DMA((3,)),
    ],
)
def _sc_edge_sum(zp_hbm, srcw_hbm, dstw_hbm, out_hbm,
                 sidx, didx, abufs, bbufs, asems, bsems, osems):
    cid = lax.axis_index("c")
    sid = lax.axis_index("s")
    wid = cid * NS + sid
    pltpu.sync_copy(srcw_hbm.at[wid], sidx)
    pltpu.sync_copy(dstw_hbm.at[wid], didx)

    pltpu.async_copy(zp_hbm.at[sidx.at[0]], abufs.at[0], asems.at[0])
    pltpu.async_copy(zp_hbm.at[didx.at[0]], bbufs.at[0], bsems.at[0])

    def out_slice(j):
        return out_hbm.at[pl.ds(wid * EPW + j * K, K)]

    def body(j, carry):
        r = lax.rem(j, 3)
        r1 = lax.rem(j + 1, 3)

        @pl.when(j >= 2)
        def _():
            pltpu.make_async_copy(abufs.at[r1], out_slice(j), osems.at[r1]).wait()

        @pl.when(j + 1 < CH)
        def _():
            pltpu.async_copy(zp_hbm.at[sidx.at[j + 1]], abufs.at[r1], asems.at[r1])
            pltpu.async_copy(zp_hbm.at[didx.at[j + 1]], bbufs.at[r1], bsems.at[r1])

        pltpu.make_async_copy(zp_hbm.at[sidx.at[j]], abufs.at[r], asems.at[r]).wait()
        pltpu.make_async_copy(zp_hbm.at[didx.at[j]], bbufs.at[r], bsems.at[r]).wait()

        ab = abufs.at[r]
        bb = bbufs.at[r]

        def row(rr, c2):
            for c in range(D // 16):
                sl = pl.ds(c * 16, 16)
                ab[rr, sl] = ab[rr, sl] + bb[rr, sl]
            return c2

        lax.fori_loop(0, K, row, 0)
        pltpu.async_copy(ab, out_slice(j), osems.at[r])
        return carry

    lax.fori_loop(0, CH, body, 0)
    pltpu.make_async_copy(abufs.at[0], out_slice(0), osems.at[lax.rem(CH - 1, 3)]).wait()
    pltpu.make_async_copy(abufs.at[0], out_slice(0), osems.at[lax.rem(CH - 2, 3)]).wait()


_BLK = 1024


def _tc_prologue_body(x_ref, w_ref, degp_ref, g_ref, dinv_ref):
    h = jnp.dot(x_ref[...], w_ref[...], preferred_element_type=jnp.float32)
    degsum = degp_ref[0, :, 0:1] + degp_ref[1, :, 0:1] + 1.0
    dinv = lax.rsqrt(degsum)
    g_ref[...] = h * dinv
    dinv_ref[...] = dinv


def _tc_prologue(x, W_gcn, degp):
    grid = (N + _BLK - 1) // _BLK
    return pl.pallas_call(
        _tc_prologue_body,
        grid=(grid,),
        in_specs=[
            pl.BlockSpec((_BLK, D), lambda i: (i, 0)),
            pl.BlockSpec((D, D), lambda i: (0, 0)),
            pl.BlockSpec((2, _BLK, D), lambda i: (0, i, 0)),
        ],
        out_specs=[
            pl.BlockSpec((_BLK, D), lambda i: (i, 0)),
            pl.BlockSpec((_BLK, 1), lambda i: (i, 0)),
        ],
        out_shape=[
            jax.ShapeDtypeStruct((N, D), jnp.float32),
            jax.ShapeDtypeStruct((N, 1), jnp.float32),
        ],
    )(x, W_gcn, degp)


def _tc_node_mlp_body(agg_ref, g_ref, dinv_ref, bg_ref, w1_ref, b1_ref,
                      w2_ref, b2_ref, z_ref):
    a = agg_ref[0] + agg_ref[1] + g_ref[...]
    x1 = jnp.maximum(a * dinv_ref[...] + bg_ref[...], 0.0)
    x2 = jnp.maximum(
        jnp.dot(x1, w1_ref[...], preferred_element_type=jnp.float32) + b1_ref[...], 0.0)
    x3 = jnp.maximum(
        jnp.dot(x2, w2_ref[...], preferred_element_type=jnp.float32) + b2_ref[...], 0.0)
    z_ref[...] = 0.5 * (
        jnp.dot(x3, w1_ref[...], preferred_element_type=jnp.float32) + b1_ref[...])


def _tc_node_mlp(agg, g, dinvc, b_gcn, W_fc1, b_fc1, W_fc2, b_fc2):
    grid = (NPAD + _BLK - 1) // _BLK
    return pl.pallas_call(
        _tc_node_mlp_body,
        grid=(grid,),
        in_specs=[
            pl.BlockSpec((2, _BLK, D), lambda i: (0, i, 0)),
            pl.BlockSpec((_BLK, D), lambda i: (i, 0)),
            pl.BlockSpec((_BLK, 1), lambda i: (i, 0)),
            pl.BlockSpec((1, D), lambda i: (0, 0)),
            pl.BlockSpec((D, D), lambda i: (0, 0)),
            pl.BlockSpec((1, D), lambda i: (0, 0)),
            pl.BlockSpec((D, D), lambda i: (0, 0)),
            pl.BlockSpec((1, D), lambda i: (0, 0)),
        ],
        out_specs=pl.BlockSpec((_BLK, D), lambda i: (i, 0)),
        out_shape=jax.ShapeDtypeStruct((NPAD, D), jnp.float32),
    )(agg, g, dinvc, b_gcn, W_fc1, b_fc1, W_fc2, b_fc2)


def _tc_edge_head_body(s_ref, w2_ref, b2_ref, wo_ref, bo_ref, o_ref):
    e1 = jnp.maximum(s_ref[...], 0.0)
    e2 = jnp.maximum(
        jnp.dot(e1.astype(jnp.bfloat16), w2_ref[...].astype(jnp.bfloat16),
                preferred_element_type=jnp.float32) + b2_ref[...], 0.0)
    sc = jnp.dot(e2.astype(jnp.bfloat16), wo_ref[...].astype(jnp.bfloat16),
                 preferred_element_type=jnp.float32) + bo_ref[...]
    s0 = sc[:, 0:1]
    s1 = sc[:, 1:2]
    m = jnp.maximum(s0, s1)
    lse = m + jnp.log(jnp.exp(s0 - m) + jnp.exp(s1 - m))
    o_ref[...] = sc - lse


def _tc_edge_head(s, W_fc2, b_fc2, W_out, b_out):
    blk = 2048
    grid = EPAD // blk
    return pl.pallas_call(
        _tc_edge_head_body,
        grid=(grid,),
        in_specs=[
            pl.BlockSpec((blk, D), lambda i: (i, 0)),
            pl.BlockSpec((D, D), lambda i: (0, 0)),
            pl.BlockSpec((1, D), lambda i: (0, 0)),
            pl.BlockSpec((D, 2), lambda i: (0, 0)),
            pl.BlockSpec((1, 2), lambda i: (0, 0)),
        ],
        out_specs=pl.BlockSpec((blk, 2), lambda i: (i, 0)),
        out_shape=jax.ShapeDtypeStruct((EPAD, 2), jnp.float32),
    )(s, W_fc2, b_fc2, W_out, b_out)


def kernel(x, edge_index, W_gcn, b_gcn, W_fc1, b_fc1, W_fc2, b_fc2, W_out, b_out):
    src = edge_index[0].astype(jnp.int32)
    dst = edge_index[1].astype(jnp.int32)
    pad = EPAD - E
    srcw = jnp.concatenate([src, jnp.zeros((pad,), jnp.int32)]).reshape(NW, CH, K)
    dstw = jnp.concatenate([dst, jnp.full((pad,), N, jnp.int32)]).reshape(NW, CH, K)
    onesD = jnp.ones((K, D), jnp.float32)
    zerosD = jnp.zeros((NPAD, D), jnp.float32)

    degp = _sc_degree(dstw, onesD, zerosD)
    g, dinvc = _tc_prologue(x, W_gcn, degp)
    agg = _sc_aggregate(g, srcw, dstw, zerosD)
    z = _tc_node_mlp(agg, g, dinvc, b_gcn.reshape(1, D), W_fc1,
                     b_fc1.reshape(1, D), W_fc2, b_fc2.reshape(1, D))
    s = _sc_edge_sum(z, srcw, dstw)
    out = _tc_edge_head(s, W_fc2, b_fc2.reshape(1, D), W_out, b_out.reshape(1, 2))
    return out[:E]


# speedup vs baseline: 6.8375x; 1.0457x over previous
import functools

import jax
import jax.numpy as jnp
from jax import lax
from jax.experimental import pallas as pl
from jax.experimental.pallas import tpu as pltpu
from jax.experimental.pallas import tpu_sc as plsc

N = 10000
E = 320000
D = 128
NPAD = 10016
NW = 32
NS = 16
K = 128
CH = 79
EPW = CH * K
EPAD = NW * EPW

_MESH = dict(core_axis_name="c", subcore_axis_name="s")


@functools.partial(
    pl.kernel,
    out_type=jax.ShapeDtypeStruct((2, NPAD, D), jnp.float32),
    mesh=plsc.VectorSubcoreMesh(**_MESH),
    scratch_types=[
        pltpu.VMEM((CH, K), jnp.int32),
        pltpu.VMEM((K, D), jnp.float32),
        pltpu.VMEM_SHARED((NPAD, D), jnp.float32),
        pltpu.SemaphoreType.DMA((2,)),
    ],
)
def _sc_degree(dstw_hbm, ones_hbm, zeros_hbm, out_hbm, idx_v, ones_v, deg_sh, sems):
    cid = lax.axis_index("c")
    sid = lax.axis_index("s")
    wid = cid * NS + sid
    pltpu.sync_copy(dstw_hbm.at[wid], idx_v)
    pltpu.sync_copy(ones_hbm, ones_v)

    @pl.when(sid == 0)
    def _():
        pltpu.sync_copy(zeros_hbm, deg_sh)

    plsc.subcore_barrier()

    def body(j, carry):
        b = lax.rem(j, 2)

        @pl.when(j >= 2)
        def _():
            pltpu.make_async_copy(ones_v, deg_sh.at[idx_v.at[j]], sems.at[b]).wait()

        pltpu.async_copy(ones_v, deg_sh.at[idx_v.at[j]], sems.at[b], add=True)
        return carry

    lax.fori_loop(0, CH, body, 0)
    pltpu.make_async_copy(ones_v, deg_sh.at[idx_v.at[0]], sems.at[0]).wait()
    pltpu.make_async_copy(ones_v, deg_sh.at[idx_v.at[0]], sems.at[1]).wait()
    plsc.subcore_barrier()

    @pl.when(sid == 0)
    def _():
        pltpu.sync_copy(deg_sh, out_hbm.at[cid])


@functools.partial(
    pl.kernel,
    out_type=jax.ShapeDtypeStruct((2, NPAD, D), jnp.float32),
    mesh=plsc.VectorSubcoreMesh(**_MESH),
    scratch_types=[
        pltpu.VMEM((2, K), jnp.int32),
        pltpu.VMEM((CH, K), jnp.int32),
        pltpu.VMEM((2, K, D), jnp.float32),
        pltpu.VMEM_SHARED((NPAD, D), jnp.float32),
        pltpu.SemaphoreType.DMA((2,)),
        pltpu.SemaphoreType.DMA((2,)),
        pltpu.SemaphoreType.DMA((2,)),
    ],
)
def _sc_aggregate(g_hbm, srcw_hbm, dstw_hbm, zeros_hbm, out_hbm,
                  sidxb, didx, bufs, agg_sh, semi, semg, sems):
    cid = lax.axis_index("c")
    sid = lax.axis_index("s")
    wid = cid * NS + sid
    pltpu.sync_copy(dstw_hbm.at[wid], didx)
    pltpu.sync_copy(srcw_hbm.at[wid, 0], sidxb.at[0])

    @pl.when(sid == 0)
    def _():
        pltpu.sync_copy(zeros_hbm, agg_sh)

    plsc.subcore_barrier()

    pltpu.async_copy(srcw_hbm.at[wid, 1], sidxb.at[1], semi.at[1])
    pltpu.async_copy(g_hbm.at[sidxb.at[0]], bufs.at[0], semg.at[0])

    def body(j, carry):
        b = lax.rem(j, 2)
        nb = lax.rem(j + 1, 2)

        @pl.when(j >= 1)
        def _():
            pltpu.make_async_copy(bufs.at[nb], agg_sh.at[didx.at[j]],
                                  sems.at[nb]).wait()

        @pl.when(j + 1 < CH)
        def _():
            pltpu.make_async_copy(srcw_hbm.at[wid, j + 1], sidxb.at[nb],
                                  semi.at[nb]).wait()
            pltpu.async_copy(g_hbm.at[sidxb.at[nb]], bufs.at[nb], semg.at[nb])

        pltpu.make_async_copy(g_hbm.at[sidxb.at[b]], bufs.at[b], semg.at[b]).wait()

        @pl.when(j + 2 < CH)
        def _():
            pltpu.async_copy(srcw_hbm.at[wid, j + 2], sidxb.at[b], semi.at[b])

        pltpu.async_copy(bufs.at[b], agg_sh.at[didx.at[j]], sems.at[b], add=True)
        return carry

    lax.fori_loop(0, CH, body, 0)
    pltpu.make_async_copy(bufs.at[0], agg_sh.at[didx.at[0]],
                          sems.at[lax.rem(CH - 1, 2)]).wait()
    plsc.subcore_barrier()

    @pl.when(sid == 0)
    def _():
        pltpu.sync_copy(agg_sh, out_hbm.at[cid])


@functools.partial(
    pl.kernel,
    out_type=jax.ShapeDtypeStruct((EPAD, D), jnp.float32),
    mesh=plsc.VectorSubcoreMesh(**_MESH),
    scratch_types=[
        pltpu.VMEM((CH, K), jnp.int32),
        pltpu.VMEM((CH, K), jnp.int32),
        pltpu.VMEM((3, K, D), jnp.float32),
        pltpu.VMEM((3, K, D), jnp.float32),
        pltpu.SemaphoreType.DMA((3,)),
        pltpu.SemaphoreType.DMA((3,)),
        pltpu.SemaphoreType.DMA((3,)),
    ],
)
def _sc_edge_sum(zp_hbm, srcw_hbm, dstw_hbm, out_hbm,
                 sidx, didx, abufs, bbufs, asems, bsems, osems):
    cid = lax.axis_index("c")
    sid = lax.axis_index("s")
    wid = cid * NS + sid
    pltpu.sync_copy(srcw_hbm.at[wid], sidx)
    pltpu.sync_copy(dstw_hbm.at[wid], didx)

    pltpu.async_copy(zp_hbm.at[sidx.at[0]], abufs.at[0], asems.at[0])
    pltpu.async_copy(zp_hbm.at[didx.at[0]], bbufs.at[0], bsems.at[0])

    def out_slice(j):
        return out_hbm.at[pl.ds(wid * EPW + j * K, K)]

    def body(j, carry):
        r = lax.rem(j, 3)
        r1 = lax.rem(j + 1, 3)

        @pl.when(j >= 2)
        def _():
            pltpu.make_async_copy(abufs.at[r1], out_slice(j), osems.at[r1]).wait()

        @pl.when(j + 1 < CH)
        def _():
            pltpu.async_copy(zp_hbm.at[sidx.at[j + 1]], abufs.at[r1], asems.at[r1])
            pltpu.async_copy(zp_hbm.at[didx.at[j + 1]], bbufs.at[r1], bsems.at[r1])

        pltpu.make_async_copy(zp_hbm.at[sidx.at[j]], abufs.at[r], asems.at[r]).wait()
        pltpu.make_async_copy(zp_hbm.at[didx.at[j]], bbufs.at[r], bsems.at[r]).wait()

        ab = abufs.at[r]
        bb = bbufs.at[r]

        def row(rr, c2):
            r2 = rr * 2
            for rk in range(2):
                for c in range(D // 16):
                    sl = pl.ds(c * 16, 16)
                    plsc.addupdate(ab.at[r2 + rk, sl], bb[r2 + rk, sl])
            return c2

        lax.fori_loop(0, K // 2, row, 0)
        pltpu.async_copy(ab, out_slice(j), osems.at[r])
        return carry

    lax.fori_loop(0, CH, body, 0)
    pltpu.make_async_copy(abufs.at[0], out_slice(0), osems.at[lax.rem(CH - 1, 3)]).wait()
    pltpu.make_async_copy(abufs.at[0], out_slice(0), osems.at[lax.rem(CH - 2, 3)]).wait()


_BLK = 1024


def _tc_prologue_body(x_ref, w_ref, degp_ref, g_ref, dinv_ref):
    h = jnp.dot(x_ref[...], w_ref[...], preferred_element_type=jnp.float32)
    degsum = degp_ref[0, :, 0:1] + degp_ref[1, :, 0:1] + 1.0
    dinv = lax.rsqrt(degsum)
    g_ref[...] = h * dinv
    dinv_ref[...] = dinv


def _tc_prologue(x, W_gcn, degp):
    grid = (N + _BLK - 1) // _BLK
    return pl.pallas_call(
        _tc_prologue_body,
        grid=(grid,),
        in_specs=[
            pl.BlockSpec((_BLK, D), lambda i: (i, 0)),
            pl.BlockSpec((D, D), lambda i: (0, 0)),
            pl.BlockSpec((2, _BLK, D), lambda i: (0, i, 0)),
        ],
        out_specs=[
            pl.BlockSpec((_BLK, D), lambda i: (i, 0)),
            pl.BlockSpec((_BLK, 1), lambda i: (i, 0)),
        ],
        out_shape=[
            jax.ShapeDtypeStruct((N, D), jnp.float32),
            jax.ShapeDtypeStruct((N, 1), jnp.float32),
        ],
    )(x, W_gcn, degp)


def _tc_node_mlp_body(agg_ref, g_ref, dinv_ref, bg_ref, w1_ref, b1_ref,
                      w2_ref, b2_ref, z_ref):
    a = agg_ref[0] + agg_ref[1] + g_ref[...]
    x1 = jnp.maximum(a * dinv_ref[...] + bg_ref[...], 0.0)
    x2 = jnp.maximum(
        jnp.dot(x1, w1_ref[...], preferred_element_type=jnp.float32) + b1_ref[...], 0.0)
    x3 = jnp.maximum(
        jnp.dot(x2, w2_ref[...], preferred_element_type=jnp.float32) + b2_ref[...], 0.0)
    z_ref[...] = 0.5 * (
        jnp.dot(x3, w1_ref[...], preferred_element_type=jnp.float32) + b1_ref[...])


def _tc_node_mlp(agg, g, dinvc, b_gcn, W_fc1, b_fc1, W_fc2, b_fc2):
    grid = (NPAD + _BLK - 1) // _BLK
    return pl.pallas_call(
        _tc_node_mlp_body,
        grid=(grid,),
        in_specs=[
            pl.BlockSpec((2, _BLK, D), lambda i: (0, i, 0)),
            pl.BlockSpec((_BLK, D), lambda i: (i, 0)),
            pl.BlockSpec((_BLK, 1), lambda i: (i, 0)),
            pl.BlockSpec((1, D), lambda i: (0, 0)),
            pl.BlockSpec((D, D), lambda i: (0, 0)),
            pl.BlockSpec((1, D), lambda i: (0, 0)),
            pl.BlockSpec((D, D), lambda i: (0, 0)),
            pl.BlockSpec((1, D), lambda i: (0, 0)),
        ],
        out_specs=pl.BlockSpec((_BLK, D), lambda i: (i, 0)),
        out_shape=jax.ShapeDtypeStruct((NPAD, D), jnp.float32),
    )(agg, g, dinvc, b_gcn, W_fc1, b_fc1, W_fc2, b_fc2)


def _tc_edge_head_body(s_ref, w2_ref, b2_ref, wo_ref, bo_ref, o_ref):
    e1 = jnp.maximum(s_ref[...], 0.0)
    e2 = jnp.maximum(
        jnp.dot(e1.astype(jnp.bfloat16), w2_ref[...].astype(jnp.bfloat16),
                preferred_element_type=jnp.float32) + b2_ref[...], 0.0)
    sc = jnp.dot(e2.astype(jnp.bfloat16), wo_ref[...].astype(jnp.bfloat16),
                 preferred_element_type=jnp.float32) + bo_ref[...]
    s0 = sc[:, 0:1]
    s1 = sc[:, 1:2]
    m = jnp.maximum(s0, s1)
    lse = m + jnp.log(jnp.exp(s0 - m) + jnp.exp(s1 - m))
    o_ref[...] = sc - lse


def _tc_edge_head(s, W_fc2, b_fc2, W_out, b_out):
    blk = 2048
    grid = EPAD // blk
    return pl.pallas_call(
        _tc_edge_head_body,
        grid=(grid,),
        in_specs=[
            pl.BlockSpec((blk, D), lambda i: (i, 0)),
            pl.BlockSpec((D, D), lambda i: (0, 0)),
            pl.BlockSpec((1, D), lambda i: (0, 0)),
            pl.BlockSpec((D, 2), lambda i: (0, 0)),
            pl.BlockSpec((1, 2), lambda i: (0, 0)),
        ],
        out_specs=pl.BlockSpec((blk, 2), lambda i: (i, 0)),
        out_shape=jax.ShapeDtypeStruct((EPAD, 2), jnp.float32),
    )(s, W_fc2, b_fc2, W_out, b_out)


def kernel(x, edge_index, W_gcn, b_gcn, W_fc1, b_fc1, W_fc2, b_fc2, W_out, b_out):
    src = edge_index[0].astype(jnp.int32)
    dst = edge_index[1].astype(jnp.int32)
    pad = EPAD - E
    srcw = jnp.concatenate([src, jnp.zeros((pad,), jnp.int32)]).reshape(NW, CH, K)
    dstw = jnp.concatenate([dst, jnp.full((pad,), N, jnp.int32)]).reshape(NW, CH, K)
    onesD = jnp.ones((K, D), jnp.float32)
    zerosD = jnp.zeros((NPAD, D), jnp.float32)

    degp = _sc_degree(dstw, onesD, zerosD)
    g, dinvc = _tc_prologue(x, W_gcn, degp)
    agg = _sc_aggregate(g, srcw, dstw, zerosD)
    z = _tc_node_mlp(agg, g, dinvc, b_gcn.reshape(1, D), W_fc1,
                     b_fc1.reshape(1, D), W_fc2, b_fc2.reshape(1, D))
    s = _sc_edge_sum(z, srcw, dstw)
    out = _tc_edge_head(s, W_fc2, b_fc2.reshape(1, D), W_out, b_out.reshape(1, 2))
    return out[:E]


# speedup vs baseline: 7.0798x; 1.0354x over previous
import functools

import jax
import jax.numpy as jnp
from jax import lax
from jax.experimental import pallas as pl
from jax.experimental.pallas import tpu as pltpu
from jax.experimental.pallas import tpu_sc as plsc

N = 10000
E = 320000
D = 128
NPAD = 10016
NW = 32
NS = 16
K = 128
CH = 79
EPW = CH * K
EPAD = NW * EPW

_MESH = dict(core_axis_name="c", subcore_axis_name="s")


@functools.partial(
    pl.kernel,
    out_type=jax.ShapeDtypeStruct((2, NPAD, D), jnp.float32),
    mesh=plsc.VectorSubcoreMesh(**_MESH),
    scratch_types=[
        pltpu.VMEM((CH, K), jnp.int32),
        pltpu.VMEM((K, D), jnp.float32),
        pltpu.VMEM_SHARED((NPAD, D), jnp.float32),
        pltpu.SemaphoreType.DMA((2,)),
    ],
)
def _sc_degree(dstw_hbm, ones_hbm, zeros_hbm, out_hbm, idx_v, ones_v, deg_sh, sems):
    cid = lax.axis_index("c")
    sid = lax.axis_index("s")
    wid = cid * NS + sid
    pltpu.sync_copy(dstw_hbm.at[wid], idx_v)
    pltpu.sync_copy(ones_hbm, ones_v)

    @pl.when(sid == 0)
    def _():
        pltpu.sync_copy(zeros_hbm, deg_sh)

    plsc.subcore_barrier()

    def body(j, carry):
        b = lax.rem(j, 2)

        @pl.when(j >= 2)
        def _():
            pltpu.make_async_copy(ones_v, deg_sh.at[idx_v.at[j]], sems.at[b]).wait()

        pltpu.async_copy(ones_v, deg_sh.at[idx_v.at[j]], sems.at[b], add=True)
        return carry

    lax.fori_loop(0, CH, body, 0)
    pltpu.make_async_copy(ones_v, deg_sh.at[idx_v.at[0]], sems.at[0]).wait()
    pltpu.make_async_copy(ones_v, deg_sh.at[idx_v.at[0]], sems.at[1]).wait()
    plsc.subcore_barrier()

    @pl.when(sid == 0)
    def _():
        pltpu.sync_copy(deg_sh, out_hbm.at[cid])


@functools.partial(
    pl.kernel,
    out_type=jax.ShapeDtypeStruct((2, NPAD, D), jnp.float32),
    mesh=plsc.VectorSubcoreMesh(**_MESH),
    scratch_types=[
        pltpu.VMEM((2, K), jnp.int32),
        pltpu.VMEM((CH, K), jnp.int32),
        pltpu.VMEM((2, K, D), jnp.float32),
        pltpu.VMEM_SHARED((NPAD, D), jnp.float32),
        pltpu.SemaphoreType.DMA((2,)),
        pltpu.SemaphoreType.DMA((2,)),
        pltpu.SemaphoreType.DMA((2,)),
    ],
)
def _sc_aggregate(g_hbm, srcw_hbm, dstw_hbm, zeros_hbm, out_hbm,
                  sidxb, didx, bufs, agg_sh, semi, semg, sems):
    cid = lax.axis_index("c")
    sid = lax.axis_index("s")
    wid = cid * NS + sid
    pltpu.sync_copy(dstw_hbm.at[wid], didx)
    pltpu.sync_copy(srcw_hbm.at[wid, 0], sidxb.at[0])

    @pl.when(sid == 0)
    def _():
        pltpu.sync_copy(zeros_hbm, agg_sh)

    plsc.subcore_barrier()

    pltpu.async_copy(srcw_hbm.at[wid, 1], sidxb.at[1], semi.at[1])
    pltpu.async_copy(g_hbm.at[sidxb.at[0]], bufs.at[0], semg.at[0])

    def body(j, carry):
        b = lax.rem(j, 2)
        nb = lax.rem(j + 1, 2)

        @pl.when(j >= 1)
        def _():
            pltpu.make_async_copy(bufs.at[nb], agg_sh.at[didx.at[j]],
                                  sems.at[nb]).wait()

        @pl.when(j + 1 < CH)
        def _():
            pltpu.make_async_copy(srcw_hbm.at[wid, j + 1], sidxb.at[nb],
                                  semi.at[nb]).wait()
            pltpu.async_copy(g_hbm.at[sidxb.at[nb]], bufs.at[nb], semg.at[nb])

        pltpu.make_async_copy(g_hbm.at[sidxb.at[b]], bufs.at[b], semg.at[b]).wait()

        @pl.when(j + 2 < CH)
        def _():
            pltpu.async_copy(srcw_hbm.at[wid, j + 2], sidxb.at[b], semi.at[b])

        pltpu.async_copy(bufs.at[b], agg_sh.at[didx.at[j]], sems.at[b], add=True)
        return carry

    lax.fori_loop(0, CH, body, 0)
    pltpu.make_async_copy(bufs.at[0], agg_sh.at[didx.at[0]],
                          sems.at[lax.rem(CH - 1, 2)]).wait()
    plsc.subcore_barrier()

    @pl.when(sid == 0)
    def _():
        pltpu.sync_copy(agg_sh, out_hbm.at[cid])


@functools.partial(
    pl.kernel,
    out_type=jax.ShapeDtypeStruct((EPAD, D), jnp.float32),
    mesh=plsc.VectorSubcoreMesh(**_MESH),
    scratch_types=[
        pltpu.VMEM((CH, K), jnp.int32),
        pltpu.VMEM((CH, K), jnp.int32),
        pltpu.VMEM((3, K, D), jnp.float32),
        pltpu.VMEM((3, K, D), jnp.float32),
        pltpu.SemaphoreType.DMA((3,)),
        pltpu.SemaphoreType.DMA((3,)),
        pltpu.SemaphoreType.DMA((3,)),
    ],
)
def _sc_edge_sum(zp_hbm, srcw_hbm, dstw_hbm, out_hbm,
                 sidx, didx, abufs, bbufs, asems, bsems, osems):
    cid = lax.axis_index("c")
    sid = lax.axis_index("s")
    wid = cid * NS + sid
    pltpu.sync_copy(srcw_hbm.at[wid], sidx)
    pltpu.sync_copy(dstw_hbm.at[wid], didx)

    pltpu.async_copy(zp_hbm.at[sidx.at[0]], abufs.at[0], asems.at[0])
    pltpu.async_copy(zp_hbm.at[didx.at[0]], bbufs.at[0], bsems.at[0])

    def out_slice(j):
        return out_hbm.at[pl.ds(wid * EPW + j * K, K)]

    def body(j, carry):
        r = lax.rem(j, 3)
        r1 = lax.rem(j + 1, 3)

        @pl.when(j >= 2)
        def _():
            pltpu.make_async_copy(abufs.at[r1], out_slice(j), osems.at[r1]).wait()

        @pl.when(j + 1 < CH)
        def _():
            pltpu.async_copy(zp_hbm.at[sidx.at[j + 1]], abufs.at[r1], asems.at[r1])
            pltpu.async_copy(zp_hbm.at[didx.at[j + 1]], bbufs.at[r1], bsems.at[r1])

        pltpu.make_async_copy(zp_hbm.at[sidx.at[j]], abufs.at[r], asems.at[r]).wait()
        pltpu.make_async_copy(zp_hbm.at[didx.at[j]], bbufs.at[r], bsems.at[r]).wait()

        ab = abufs.at[r]
        bb = bbufs.at[r]

        def row(rr, c2):
            r2 = rr * 2
            for rk in range(2):
                for c in range(D // 16):
                    sl = pl.ds(c * 16, 16)
                    plsc.addupdate(ab.at[r2 + rk, sl], bb[r2 + rk, sl])
            return c2

        lax.fori_loop(0, K // 2, row, 0)
        pltpu.async_copy(ab, out_slice(j), osems.at[r])
        return carry

    lax.fori_loop(0, CH, body, 0)
    pltpu.make_async_copy(abufs.at[0], out_slice(0), osems.at[lax.rem(CH - 1, 3)]).wait()
    pltpu.make_async_copy(abufs.at[0], out_slice(0), osems.at[lax.rem(CH - 2, 3)]).wait()


_BLK = 1024


def _tc_prologue_body(x_ref, w_ref, degp_ref, g_ref, dinv_ref):
    h = jnp.dot(x_ref[...], w_ref[...], preferred_element_type=jnp.float32)
    degsum = degp_ref[0, :, 0:1] + degp_ref[1, :, 0:1] + 1.0
    dinv = lax.rsqrt(degsum)
    g_ref[...] = h * dinv
    dinv_ref[...] = dinv


def _tc_prologue(x, W_gcn, degp):
    grid = (N + _BLK - 1) // _BLK
    return pl.pallas_call(
        _tc_prologue_body,
        grid=(grid,),
        in_specs=[
            pl.BlockSpec((_BLK, D), lambda i: (i, 0)),
            pl.BlockSpec((D, D), lambda i: (0, 0)),
            pl.BlockSpec((2, _BLK, D), lambda i: (0, i, 0)),
        ],
        out_specs=[
            pl.BlockSpec((_BLK, D), lambda i: (i, 0)),
            pl.BlockSpec((_BLK, 1), lambda i: (i, 0)),
        ],
        out_shape=[
            jax.ShapeDtypeStruct((N, D), jnp.float32),
            jax.ShapeDtypeStruct((N, 1), jnp.float32),
        ],
    )(x, W_gcn, degp)


def _tc_node_mlp_body(agg_ref, g_ref, dinv_ref, bg_ref, w1_ref, b1_ref,
                      w2_ref, b2_ref, z_ref):
    a = agg_ref[0] + agg_ref[1] + g_ref[...]
    x1 = jnp.maximum(a * dinv_ref[...] + bg_ref[...], 0.0)
    x2 = jnp.maximum(
        jnp.dot(x1, w1_ref[...], preferred_element_type=jnp.float32) + b1_ref[...], 0.0)
    x3 = jnp.maximum(
        jnp.dot(x2, w2_ref[...], preferred_element_type=jnp.float32) + b2_ref[...], 0.0)
    z_ref[...] = 0.5 * (
        jnp.dot(x3, w1_ref[...], preferred_element_type=jnp.float32) + b1_ref[...])


def _tc_node_mlp(agg, g, dinvc, b_gcn, W_fc1, b_fc1, W_fc2, b_fc2):
    grid = (NPAD + _BLK - 1) // _BLK
    return pl.pallas_call(
        _tc_node_mlp_body,
        grid=(grid,),
        in_specs=[
            pl.BlockSpec((2, _BLK, D), lambda i: (0, i, 0)),
            pl.BlockSpec((_BLK, D), lambda i: (i, 0)),
            pl.BlockSpec((_BLK, 1), lambda i: (i, 0)),
            pl.BlockSpec((1, D), lambda i: (0, 0)),
            pl.BlockSpec((D, D), lambda i: (0, 0)),
            pl.BlockSpec((1, D), lambda i: (0, 0)),
            pl.BlockSpec((D, D), lambda i: (0, 0)),
            pl.BlockSpec((1, D), lambda i: (0, 0)),
        ],
        out_specs=pl.BlockSpec((_BLK, D), lambda i: (i, 0)),
        out_shape=jax.ShapeDtypeStruct((NPAD, D), jnp.float32),
    )(agg, g, dinvc, b_gcn, W_fc1, b_fc1, W_fc2, b_fc2)


def _tc_edge_head_body(s_ref, w2_ref, b2_ref, wo_ref, bo_ref, o_ref):
    e1 = jnp.maximum(s_ref[...], 0.0)
    e2 = jnp.maximum(
        jnp.dot(e1.astype(jnp.bfloat16), w2_ref[...].astype(jnp.bfloat16),
                preferred_element_type=jnp.float32) + b2_ref[...], 0.0)
    sc = jnp.dot(e2.astype(jnp.bfloat16), wo_ref[...].astype(jnp.bfloat16),
                 preferred_element_type=jnp.float32) + bo_ref[...]
    s0 = sc[:, 0:1]
    s1 = sc[:, 1:2]
    m = jnp.maximum(s0, s1)
    lse = m + jnp.log(jnp.exp(s0 - m) + jnp.exp(s1 - m))
    o_ref[...] = sc - lse


def _tc_edge_head(s, W_fc2, b_fc2, W_out, b_out):
    blk = 4096
    grid = EPAD // blk
    return pl.pallas_call(
        _tc_edge_head_body,
        grid=(grid,),
        in_specs=[
            pl.BlockSpec((blk, D), lambda i: (i, 0)),
            pl.BlockSpec((D, D), lambda i: (0, 0)),
            pl.BlockSpec((1, D), lambda i: (0, 0)),
            pl.BlockSpec((D, 2), lambda i: (0, 0)),
            pl.BlockSpec((1, 2), lambda i: (0, 0)),
        ],
        out_specs=pl.BlockSpec((blk, 2), lambda i: (i, 0)),
        out_shape=jax.ShapeDtypeStruct((EPAD, 2), jnp.float32),
    )(s, W_fc2, b_fc2, W_out, b_out)


def kernel(x, edge_index, W_gcn, b_gcn, W_fc1, b_fc1, W_fc2, b_fc2, W_out, b_out):
    src = edge_index[0].astype(jnp.int32)
    dst = edge_index[1].astype(jnp.int32)
    pad = EPAD - E
    srcw = jnp.concatenate([src, jnp.zeros((pad,), jnp.int32)]).reshape(NW, CH, K)
    dstw = jnp.concatenate([dst, jnp.full((pad,), N, jnp.int32)]).reshape(NW, CH, K)
    onesD = jnp.ones((K, D), jnp.float32)
    zerosD = jnp.zeros((NPAD, D), jnp.float32)

    degp = _sc_degree(dstw, onesD, zerosD)
    g, dinvc = _tc_prologue(x, W_gcn, degp)
    agg = _sc_aggregate(g, srcw, dstw, zerosD)
    z = _tc_node_mlp(agg, g, dinvc, b_gcn.reshape(1, D), W_fc1,
                     b_fc1.reshape(1, D), W_fc2, b_fc2.reshape(1, D))
    s = _sc_edge_sum(z, srcw, dstw)
    out = _tc_edge_head(s, W_fc2, b_fc2.reshape(1, D), W_out, b_out.reshape(1, 2))
    return out[:E]


# speedup vs baseline: 7.1497x; 1.0099x over previous
import functools

import jax
import jax.numpy as jnp
from jax import lax
from jax.experimental import pallas as pl
from jax.experimental.pallas import tpu as pltpu
from jax.experimental.pallas import tpu_sc as plsc

N = 10000
E = 320000
D = 128
NPAD = 10016
NW = 32
NS = 16
K = 128
CH = 79
EPW = CH * K
EPAD = NW * EPW

_MESH = dict(core_axis_name="c", subcore_axis_name="s")


@functools.partial(
    pl.kernel,
    out_type=jax.ShapeDtypeStruct((2, NPAD, D), jnp.float32),
    mesh=plsc.VectorSubcoreMesh(**_MESH),
    scratch_types=[
        pltpu.VMEM((CH, K), jnp.int32),
        pltpu.VMEM((K, D), jnp.float32),
        pltpu.VMEM_SHARED((NPAD, D), jnp.float32),
        pltpu.SemaphoreType.DMA((2,)),
    ],
)
def _sc_degree(dstw_hbm, ones_hbm, zeros_hbm, out_hbm, idx_v, ones_v, deg_sh, sems):
    cid = lax.axis_index("c")
    sid = lax.axis_index("s")
    wid = cid * NS + sid
    pltpu.sync_copy(dstw_hbm.at[wid], idx_v)
    pltpu.sync_copy(ones_hbm, ones_v)

    @pl.when(sid == 0)
    def _():
        pltpu.sync_copy(zeros_hbm, deg_sh)

    plsc.subcore_barrier()

    def body(j, carry):
        b = lax.rem(j, 2)

        @pl.when(j >= 2)
        def _():
            pltpu.make_async_copy(ones_v, deg_sh.at[idx_v.at[j]], sems.at[b]).wait()

        pltpu.async_copy(ones_v, deg_sh.at[idx_v.at[j]], sems.at[b], add=True)
        return carry

    lax.fori_loop(0, CH, body, 0)
    pltpu.make_async_copy(ones_v, deg_sh.at[idx_v.at[0]], sems.at[0]).wait()
    pltpu.make_async_copy(ones_v, deg_sh.at[idx_v.at[0]], sems.at[1]).wait()
    plsc.subcore_barrier()

    @pl.when(sid == 0)
    def _():
        pltpu.sync_copy(deg_sh, out_hbm.at[cid])


@functools.partial(
    pl.kernel,
    out_type=jax.ShapeDtypeStruct((2, NPAD, D), jnp.float32),
    mesh=plsc.VectorSubcoreMesh(**_MESH),
    scratch_types=[
        pltpu.VMEM((2, K), jnp.int32),
        pltpu.VMEM((CH, K), jnp.int32),
        pltpu.VMEM((2, K, D), jnp.float32),
        pltpu.VMEM_SHARED((NPAD, D), jnp.float32),
        pltpu.SemaphoreType.DMA((2,)),
        pltpu.SemaphoreType.DMA((2,)),
        pltpu.SemaphoreType.DMA((2,)),
    ],
)
def _sc_aggregate(g_hbm, srcw_hbm, dstw_hbm, zeros_hbm, out_hbm,
                  sidxb, didx, bufs, agg_sh, semi, semg, sems):
    cid = lax.axis_index("c")
    sid = lax.axis_index("s")
    wid = cid * NS + sid
    pltpu.sync_copy(dstw_hbm.at[wid], didx)
    pltpu.sync_copy(srcw_hbm.at[wid, 0], sidxb.at[0])

    @pl.when(sid == 0)
    def _():
        pltpu.sync_copy(zeros_hbm, agg_sh)

    plsc.subcore_barrier()

    pltpu.async_copy(srcw_hbm.at[wid, 1], sidxb.at[1], semi.at[1])
    pltpu.async_copy(g_hbm.at[sidxb.at[0]], bufs.at[0], semg.at[0])

    def body(j, carry):
        b = lax.rem(j, 2)
        nb = lax.rem(j + 1, 2)

        @pl.when(j >= 1)
        def _():
            pltpu.make_async_copy(bufs.at[nb], agg_sh.at[didx.at[j]],
                                  sems.at[nb]).wait()

        @pl.when(j + 1 < CH)
        def _():
            pltpu.make_async_copy(srcw_hbm.at[wid, j + 1], sidxb.at[nb],
                                  semi.at[nb]).wait()
            pltpu.async_copy(g_hbm.at[sidxb.at[nb]], bufs.at[nb], semg.at[nb])

        pltpu.make_async_copy(g_hbm.at[sidxb.at[b]], bufs.at[b], semg.at[b]).wait()

        @pl.when(j + 2 < CH)
        def _():
            pltpu.async_copy(srcw_hbm.at[wid, j + 2], sidxb.at[b], semi.at[b])

        pltpu.async_copy(bufs.at[b], agg_sh.at[didx.at[j]], sems.at[b], add=True)
        return carry

    lax.fori_loop(0, CH, body, 0)
    pltpu.make_async_copy(bufs.at[0], agg_sh.at[didx.at[0]],
                          sems.at[lax.rem(CH - 1, 2)]).wait()
    plsc.subcore_barrier()

    @pl.when(sid == 0)
    def _():
        pltpu.sync_copy(agg_sh, out_hbm.at[cid])


@functools.partial(
    pl.kernel,
    out_type=jax.ShapeDtypeStruct((EPAD, D), jnp.float32),
    mesh=plsc.VectorSubcoreMesh(**_MESH),
    scratch_types=[
        pltpu.VMEM((CH, K), jnp.int32),
        pltpu.VMEM((CH, K), jnp.int32),
        pltpu.VMEM((3, K, D), jnp.float32),
        pltpu.VMEM((3, K, D), jnp.float32),
        pltpu.SemaphoreType.DMA((3,)),
        pltpu.SemaphoreType.DMA((3,)),
        pltpu.SemaphoreType.DMA((3,)),
    ],
)
def _sc_edge_sum(zp_hbm, srcw_hbm, dstw_hbm, out_hbm,
                 sidx, didx, abufs, bbufs, asems, bsems, osems):
    cid = lax.axis_index("c")
    sid = lax.axis_index("s")
    wid = cid * NS + sid
    pltpu.sync_copy(srcw_hbm.at[wid], sidx)
    pltpu.sync_copy(dstw_hbm.at[wid], didx)

    pltpu.async_copy(zp_hbm.at[sidx.at[0]], abufs.at[0], asems.at[0])
    pltpu.async_copy(zp_hbm.at[didx.at[0]], bbufs.at[0], bsems.at[0])

    def out_slice(j):
        return out_hbm.at[pl.ds(wid * EPW + j * K, K)]

    def body(j, carry):
        r = lax.rem(j, 3)
        r1 = lax.rem(j + 1, 3)

        @pl.when(j >= 2)
        def _():
            pltpu.make_async_copy(abufs.at[r1], out_slice(j), osems.at[r1]).wait()

        @pl.when(j + 1 < CH)
        def _():
            pltpu.async_copy(zp_hbm.at[sidx.at[j + 1]], abufs.at[r1], asems.at[r1])
            pltpu.async_copy(zp_hbm.at[didx.at[j + 1]], bbufs.at[r1], bsems.at[r1])

        pltpu.make_async_copy(zp_hbm.at[sidx.at[j]], abufs.at[r], asems.at[r]).wait()
        pltpu.make_async_copy(zp_hbm.at[didx.at[j]], bbufs.at[r], bsems.at[r]).wait()

        ab = abufs.at[r]
        bb = bbufs.at[r]

        def row(rr, c2):
            r2 = rr * 2
            for rk in range(2):
                for c in range(D // 16):
                    sl = pl.ds(c * 16, 16)
                    plsc.addupdate(ab.at[r2 + rk, sl], bb[r2 + rk, sl])
            return c2

        lax.fori_loop(0, K // 2, row, 0)
        pltpu.async_copy(ab, out_slice(j), osems.at[r])
        return carry

    lax.fori_loop(0, CH, body, 0)
    pltpu.make_async_copy(abufs.at[0], out_slice(0), osems.at[lax.rem(CH - 1, 3)]).wait()
    pltpu.make_async_copy(abufs.at[0], out_slice(0), osems.at[lax.rem(CH - 2, 3)]).wait()


_BLK = 1024


def _tc_prologue_body(x_ref, w_ref, degp_ref, g_ref, dinv_ref):
    h = jnp.dot(x_ref[...], w_ref[...], preferred_element_type=jnp.float32)
    degsum = degp_ref[0, :, 0:1] + degp_ref[1, :, 0:1] + 1.0
    dinv = lax.rsqrt(degsum)
    g_ref[...] = h * dinv
    dinv_ref[...] = dinv


def _tc_prologue(x, W_gcn, degp):
    grid = (N + _BLK - 1) // _BLK
    return pl.pallas_call(
        _tc_prologue_body,
        grid=(grid,),
        in_specs=[
            pl.BlockSpec((_BLK, D), lambda i: (i, 0)),
            pl.BlockSpec((D, D), lambda i: (0, 0)),
            pl.BlockSpec((2, _BLK, D), lambda i: (0, i, 0)),
        ],
        out_specs=[
            pl.BlockSpec((_BLK, D), lambda i: (i, 0)),
            pl.BlockSpec((_BLK, 1), lambda i: (i, 0)),
        ],
        out_shape=[
            jax.ShapeDtypeStruct((N, D), jnp.float32),
            jax.ShapeDtypeStruct((N, 1), jnp.float32),
        ],
    )(x, W_gcn, degp)


def _tc_node_mlp_body(agg_ref, g_ref, dinv_ref, bg_ref, w1_ref, b1_ref,
                      w2_ref, b2_ref, z_ref):
    a = agg_ref[0] + agg_ref[1] + g_ref[...]
    x1 = jnp.maximum(a * dinv_ref[...] + bg_ref[...], 0.0)
    x2 = jnp.maximum(
        jnp.dot(x1, w1_ref[...], preferred_element_type=jnp.float32) + b1_ref[...], 0.0)
    x3 = jnp.maximum(
        jnp.dot(x2, w2_ref[...], preferred_element_type=jnp.float32) + b2_ref[...], 0.0)
    z_ref[...] = 0.5 * (
        jnp.dot(x3, w1_ref[...], preferred_element_type=jnp.float32) + b1_ref[...])


def _tc_node_mlp(agg, g, dinvc, b_gcn, W_fc1, b_fc1, W_fc2, b_fc2):
    grid = (NPAD + _BLK - 1) // _BLK
    return pl.pallas_call(
        _tc_node_mlp_body,
        grid=(grid,),
        in_specs=[
            pl.BlockSpec((2, _BLK, D), lambda i: (0, i, 0)),
            pl.BlockSpec((_BLK, D), lambda i: (i, 0)),
            pl.BlockSpec((_BLK, 1), lambda i: (i, 0)),
            pl.BlockSpec((1, D), lambda i: (0, 0)),
            pl.BlockSpec((D, D), lambda i: (0, 0)),
            pl.BlockSpec((1, D), lambda i: (0, 0)),
            pl.BlockSpec((D, D), lambda i: (0, 0)),
            pl.BlockSpec((1, D), lambda i: (0, 0)),
        ],
        out_specs=pl.BlockSpec((_BLK, D), lambda i: (i, 0)),
        out_shape=jax.ShapeDtypeStruct((NPAD, D), jnp.float32),
    )(agg, g, dinvc, b_gcn, W_fc1, b_fc1, W_fc2, b_fc2)


def _tc_edge_head_body(s_ref, w2_ref, b2_ref, wo_ref, bo_ref, o_ref):
    e1 = jnp.maximum(s_ref[...], 0.0)
    e2 = jnp.maximum(
        jnp.dot(e1.astype(jnp.bfloat16), w2_ref[...].astype(jnp.bfloat16),
                preferred_element_type=jnp.float32) + b2_ref[...], 0.0)
    sc = jnp.dot(e2.astype(jnp.bfloat16), wo_ref[...].astype(jnp.bfloat16),
                 preferred_element_type=jnp.float32) + bo_ref[...]
    s0 = sc[:, 0:1]
    s1 = sc[:, 1:2]
    m = jnp.maximum(s0, s1)
    lse = m + jnp.log(jnp.exp(s0 - m) + jnp.exp(s1 - m))
    o_ref[...] = sc - lse


def _tc_edge_head(s, W_fc2, b_fc2, W_out, b_out):
    blk = 8192
    grid = (EPAD + blk - 1) // blk
    return pl.pallas_call(
        _tc_edge_head_body,
        grid=(grid,),
        in_specs=[
            pl.BlockSpec((blk, D), lambda i: (i, 0)),
            pl.BlockSpec((D, D), lambda i: (0, 0)),
            pl.BlockSpec((1, D), lambda i: (0, 0)),
            pl.BlockSpec((D, 2), lambda i: (0, 0)),
            pl.BlockSpec((1, 2), lambda i: (0, 0)),
        ],
        out_specs=pl.BlockSpec((blk, 2), lambda i: (i, 0)),
        out_shape=jax.ShapeDtypeStruct((EPAD, 2), jnp.float32),
    )(s, W_fc2, b_fc2, W_out, b_out)


def kernel(x, edge_index, W_gcn, b_gcn, W_fc1, b_fc1, W_fc2, b_fc2, W_out, b_out):
    src = edge_index[0].astype(jnp.int32)
    dst = edge_index[1].astype(jnp.int32)
    pad = EPAD - E
    srcw = jnp.concatenate([src, jnp.zeros((pad,), jnp.int32)]).reshape(NW, CH, K)
    dstw = jnp.concatenate([dst, jnp.full((pad,), N, jnp.int32)]).reshape(NW, CH, K)
    onesD = jnp.ones((K, D), jnp.float32)
    zerosD = jnp.zeros((NPAD, D), jnp.float32)

    degp = _sc_degree(dstw, onesD, zerosD)
    g, dinvc = _tc_prologue(x, W_gcn, degp)
    agg = _sc_aggregate(g, srcw, dstw, zerosD)
    z = _tc_node_mlp(agg, g, dinvc, b_gcn.reshape(1, D), W_fc1,
                     b_fc1.reshape(1, D), W_fc2, b_fc2.reshape(1, D))
    s = _sc_edge_sum(z, srcw, dstw)
    out = _tc_edge_head(s, W_fc2, b_fc2.reshape(1, D), W_out, b_out.reshape(1, 2))
    return out[:E]
